# Initial kernel scaffold; baseline (speedup 1.0000x reference)
import jax
import jax.numpy as jnp
from jax.experimental import pallas as pl


def kernel(x0, edge_index0, batch0, x1, edge_index1, batch1, Wl1, bl1, Wr1, Wl2, bl2, Wr2, Wc1, bc1, Wc2, bc2):
    raise NotImplementedError("write your pallas kernel here")


# speedup vs baseline: 3.4247x; 3.4247x over previous
import functools

import jax
import jax.numpy as jnp
from jax import lax
from jax.experimental import pallas as pl
from jax.experimental.pallas import tpu as pltpu
from jax.experimental.pallas import tpu_sc as plsc

N = 10000
E = 320000
D = 128
G = 16
GP = 32

NSUB = 16
NPT = 640
NP = NSUB * NPT
CH = 64
ECH = 314
EPT = ECH * CH
EP = NSUB * EPT
NCH = NPT // CH


def _sc_passes(x0, x1, s0, d0, s1, d1, b0, b1, z128, ones128):
    mesh = plsc.VectorSubcoreMesh(core_axis_name="c", subcore_axis_name="s")
    f32 = jnp.float32
    outs = (
        jax.ShapeDtypeStruct((NP, D), f32),
        jax.ShapeDtypeStruct((NP, D), f32),
        jax.ShapeDtypeStruct((GP, D), f32),
        jax.ShapeDtypeStruct((GP, D), f32),
        jax.ShapeDtypeStruct((GP, D), f32),
        jax.ShapeDtypeStruct((GP, D), f32),
        jax.ShapeDtypeStruct((GP, D), f32),
        jax.ShapeDtypeStruct((GP, D), f32),
        jax.ShapeDtypeStruct((GP, D), f32),
        jax.ShapeDtypeStruct((GP, D), f32),
        jax.ShapeDtypeStruct((GP, D), f32),
        jax.ShapeDtypeStruct((GP, D), f32),
    )

    @functools.partial(
        pl.kernel,
        mesh=mesh,
        out_type=outs,
        scratch_types=[
            pltpu.VMEM((1, CH), jnp.int32),
            pltpu.VMEM((1, CH), jnp.int32),
            pltpu.VMEM((CH, D), f32),
            pltpu.VMEM((NPT // 8, D), f32),
            pltpu.VMEM((CH, D), f32),
            pltpu.VMEM((CH, D), f32),
            pltpu.VMEM((NCH, CH), jnp.int32),
            pltpu.VMEM_SHARED((NP, D), f32),
            pltpu.VMEM_SHARED((GP, D), f32),
            pltpu.VMEM_SHARED((GP, D), f32),
            pltpu.VMEM_SHARED((GP, D), f32),
            pltpu.VMEM_SHARED((GP, D), f32),
            pltpu.VMEM_SHARED((GP, D), f32),
            pltpu.SemaphoreType.DMA,
        ],
    )
    def body(x0r, x1r, s0r, d0r, s1r, d1r, b0r, b1r, z128r, ones128r,
             y0r, y1r, px0r, py0r, pw0r, pm0r, nc0r,
             px1r, py1r, pw1r, pm1r, nc1r,
             srcv, dstv, rows, cntv, mv, onesv, bv,
             acc, pxS, pyS, pwS, pmS, ncS, sem):
        c = lax.axis_index("c")
        s = lax.axis_index("s")
        r0 = s * NPT

        def cnt_vec(r):
            return cntv[r // 8, pl.ds((r % 8) * 16, 16)]

        def divide_rows():
            def rowfix(i, carry):
                dv = jnp.maximum(cnt_vec(carry + i), 1.0)
                for cc in range(D // 16):
                    rows[i, pl.ds(cc * 16, 16)] = rows[i, pl.ds(cc * 16, 16)] / dv
                return carry
            return rowfix

        def zero_acc_slice():
            def zrow(k, carry):
                pltpu.sync_copy(rows, acc.at[pl.ds(r0 + k * CH, CH)])
                return carry
            lax.fori_loop(0, NCH, zrow, 0)

        pltpu.sync_copy(ones128r, onesv)
        pltpu.sync_copy(z128r, rows)
        zero_acc_slice()

        @pl.when(s == 0)
        def _():
            pltpu.sync_copy(rows.at[pl.ds(0, GP)], pxS)
            pltpu.sync_copy(rows.at[pl.ds(0, GP)], pyS)
            pltpu.sync_copy(rows.at[pl.ds(0, GP)], pwS)
            pltpu.sync_copy(rows.at[pl.ds(0, GP)], pmS)
            pltpu.sync_copy(rows.at[pl.ds(0, GP)], ncS)

        @pl.when(c == 0)
        def _():
            pltpu.sync_copy(b0r.at[s], bv)

        @pl.when(c == 1)
        def _():
            pltpu.sync_copy(b1r.at[s], bv)

        plsc.subcore_barrier()

        pltpu.sync_copy(ones128r, rows)

        def p0_loop(dr):
            def p0(j, carry):
                pltpu.sync_copy(dr.at[s, j], dstv.at[0])
                pltpu.sync_copy(rows, acc.at[dstv.at[0]], add=True)
                return carry
            lax.fori_loop(0, ECH, p0, 0)

        @pl.when(c == 0)
        def _():
            p0_loop(d0r)

        @pl.when(c == 1)
        def _():
            p0_loop(d1r)

        plsc.subcore_barrier()

        def pext(k, carry):
            pltpu.sync_copy(acc.at[pl.ds(r0 + k * CH, CH)], rows)

            def crow(i, carry2):
                r = carry2 + i
                cntv[r // 8, pl.ds((r % 8) * 16, 16)] = rows[i, pl.ds(0, 16)]
                return carry2
            lax.fori_loop(0, CH, crow, k * CH)
            return carry
        lax.fori_loop(0, NCH, pext, 0)
        pltpu.sync_copy(z128r, rows)
        zero_acc_slice()
        plsc.subcore_barrier()

        def p1_loop(xr, sr, dr):
            def p1(j, carry):
                pltpu.sync_copy(sr.at[s, j], srcv.at[0])
                pltpu.sync_copy(dr.at[s, j], dstv.at[0])
                pltpu.async_copy(xr.at[srcv.at[0]], rows, sem).wait()
                pltpu.sync_copy(rows, acc.at[dstv.at[0]], add=True)
                return carry
            lax.fori_loop(0, ECH, p1, 0)

        @pl.when(c == 0)
        def _():
            p1_loop(x0r, s0r, d0r)

        @pl.when(c == 1)
        def _():
            p1_loop(x1r, s1r, d1r)

        plsc.subcore_barrier()

        def p2_loop(xr, yr):
            def p2(k, carry):
                row0 = r0 + k * CH
                pltpu.sync_copy(acc.at[pl.ds(row0, CH)], rows)
                lax.fori_loop(0, CH, divide_rows(), k * CH)

                def mrow(i, carry2):
                    mval = jnp.where(cnt_vec(carry2 + i) > 0.0, 1.0, 0.0)
                    for cc in range(D // 16):
                        mv[i, pl.ds(cc * 16, 16)] = mval
                    return carry2
                lax.fori_loop(0, CH, mrow, k * CH)
                pltpu.sync_copy(rows, yr.at[pl.ds(row0, CH)])
                pltpu.sync_copy(rows, pyS.at[bv.at[k]], add=True)
                pltpu.sync_copy(mv, pmS.at[bv.at[k]], add=True)
                pltpu.sync_copy(onesv, ncS.at[bv.at[k]], add=True)
                pltpu.sync_copy(xr.at[pl.ds(row0, CH)], rows)
                pltpu.sync_copy(rows, pxS.at[bv.at[k]], add=True)
                return carry
            lax.fori_loop(0, NCH, p2, 0)

        @pl.when(c == 0)
        def _():
            p2_loop(x0r, y0r)

        @pl.when(c == 1)
        def _():
            p2_loop(x1r, y1r)

        pltpu.sync_copy(z128r, rows)
        zero_acc_slice()
        plsc.subcore_barrier()

        def p3_loop(sr, dr, yr):
            def p3(j, carry):
                pltpu.sync_copy(sr.at[s, j], srcv.at[0])
                pltpu.sync_copy(dr.at[s, j], dstv.at[0])
                pltpu.async_copy(yr.at[srcv.at[0]], rows, sem).wait()
                pltpu.sync_copy(rows, acc.at[dstv.at[0]], add=True)
                return carry
            lax.fori_loop(0, ECH, p3, 0)

        @pl.when(c == 0)
        def _():
            p3_loop(s0r, d0r, y0r)

        @pl.when(c == 1)
        def _():
            p3_loop(s1r, d1r, y1r)

        plsc.subcore_barrier()

        def p4(k, carry):
            row0 = r0 + k * CH
            pltpu.sync_copy(acc.at[pl.ds(row0, CH)], rows)
            lax.fori_loop(0, CH, divide_rows(), k * CH)
            pltpu.sync_copy(rows, pwS.at[bv.at[k]], add=True)
            return carry
        lax.fori_loop(0, NCH, p4, 0)
        plsc.subcore_barrier()

        def publish(pxo, pyo, pwo, pmo, nco):
            pltpu.sync_copy(pxS, rows.at[pl.ds(0, GP)])
            pltpu.sync_copy(rows.at[pl.ds(0, GP)], pxo)
            pltpu.sync_copy(pyS, rows.at[pl.ds(0, GP)])
            pltpu.sync_copy(rows.at[pl.ds(0, GP)], pyo)
            pltpu.sync_copy(pwS, rows.at[pl.ds(0, GP)])
            pltpu.sync_copy(rows.at[pl.ds(0, GP)], pwo)
            pltpu.sync_copy(pmS, rows.at[pl.ds(0, GP)])
            pltpu.sync_copy(rows.at[pl.ds(0, GP)], pmo)
            pltpu.sync_copy(ncS, rows.at[pl.ds(0, GP)])
            pltpu.sync_copy(rows.at[pl.ds(0, GP)], nco)

        @pl.when((s == 0) & (c == 0))
        def _():
            publish(px0r, py0r, pw0r, pm0r, nc0r)

        @pl.when((s == 0) & (c == 1))
        def _():
            publish(px1r, py1r, pw1r, pm1r, nc1r)

    return body(x0, x1, s0, d0, s1, d1, b0, b1, z128, ones128)


def _mm(a, b):
    return lax.dot_general(a, b, (((1,), (1,)), ((), ())),
                           precision=lax.Precision.HIGHEST,
                           preferred_element_type=jnp.float32)


def _sigmoid(x):
    return 1.0 / (1.0 + jnp.exp(-x))


def _tc_dense(px0, py0, pw0, pm0, nc0, px1, py1, pw1, pm1, nc1,
              Wl1, bl1, Wr1, Wl2, bl2, Wr2, Wc1, bc1, Wc2, bc2):
    def body(px0r, py0r, pw0r, pm0r, nc0r, px1r, py1r, pw1r, pm1r, nc1r,
             Wl1r, bl1r, Wr1r, Wl2r, bl2r, Wr2r, Wc1r, bc1r, Wc2r, bc2r,
             outr):
        A1, B1 = Wl1r[...], bl1r[...]
        R1 = Wr1r[...]
        A2, B2, R2 = Wl2r[...], bl2r[...], Wr2r[...]

        def graph(pxr, pyr, pwr, pmr, ncr):
            nc = ncr[...][:G, 0:1]
            inv = 1.0 / jnp.maximum(nc, 1.0)
            u = jnp.where(nc > 0.0, 1.0, 0.0)
            px = pxr[...][:G, :] * inv
            py = pyr[...][:G, :] * inv
            pw = pwr[...][:G, :] * inv
            pm = pmr[...][:G, 0:1] * inv
            Pf = _mm(py, A1) + _mm(px, R1) + u * B1
            Pz = _mm(pw, A1) + _mm(py, R1) + pm * B1
            return _mm(Pz, A2) + _mm(Pf, R2) + u * B2

        e0 = graph(px0r, py0r, pw0r, pm0r, nc0r)
        e1 = graph(px1r, py1r, pw1r, pm1r, nc1r)
        comb = jnp.concatenate([e0, e1], axis=1)
        h = _sigmoid(_mm(comb, Wc1r[...]) + bc1r[...])
        o = jnp.sum(h * Wc2r[...], axis=1, keepdims=True) + bc2r[...][0, 0]
        outr[...] = _sigmoid(o)

    return pl.pallas_call(
        body,
        out_shape=jax.ShapeDtypeStruct((G, 1), jnp.float32),
    )(px0, py0, pw0, pm0, nc0, px1, py1, pw1, pm1, nc1,
      Wl1, bl1.reshape(1, -1), Wr1, Wl2, bl2.reshape(1, -1), Wr2,
      Wc1, bc1.reshape(1, -1), Wc2, bc2.reshape(1, 1))


def _prep_graph(x, edge_index, batch):
    src = edge_index[0].astype(jnp.int32)
    dst = edge_index[1].astype(jnp.int32)
    srcp = jnp.concatenate([src, jnp.zeros((EP - E,), jnp.int32)]).reshape(NSUB, ECH, CH)
    dstp = jnp.concatenate([dst, jnp.full((EP - E,), N, jnp.int32)]).reshape(NSUB, ECH, CH)
    xp = jnp.concatenate([x, jnp.zeros((NP - N, D), x.dtype)], axis=0)
    bp = jnp.concatenate([batch.astype(jnp.int32),
                          jnp.full((NP - N,), G, jnp.int32)]).reshape(NSUB, NCH, CH)
    return xp, srcp, dstp, bp


def kernel(x0, edge_index0, batch0, x1, edge_index1, batch1,
           Wl1, bl1, Wr1, Wl2, bl2, Wr2, Wc1, bc1, Wc2, bc2):
    x0p, s0, d0, b0 = _prep_graph(x0, edge_index0, batch0)
    x1p, s1, d1, b1 = _prep_graph(x1, edge_index1, batch1)
    z128 = jnp.zeros((CH, D), jnp.float32)
    ones128 = jnp.ones((CH, D), jnp.float32)

    hbm = lambda a: pltpu.with_memory_space_constraint(a, pltpu.HBM)
    (_, _, px0, py0, pw0, pm0, nc0, px1, py1, pw1, pm1, nc1) = _sc_passes(
        hbm(x0p), hbm(x1p), hbm(s0), hbm(d0), hbm(s1), hbm(d1),
        hbm(b0), hbm(b1), hbm(z128), hbm(ones128))

    prob = _tc_dense(px0, py0, pw0, pm0, nc0, px1, py1, pw1, pm1, nc1,
                     Wl1, bl1, Wr1, Wl2, bl2, Wr2, Wc1, bc1, Wc2, bc2)
    return jnp.squeeze(prob, axis=-1)


# speedup vs baseline: 5.3849x; 1.5724x over previous
import functools

import jax
import jax.numpy as jnp
from jax import lax
from jax.experimental import pallas as pl
from jax.experimental.pallas import tpu as pltpu
from jax.experimental.pallas import tpu_sc as plsc

N = 10000
E = 320000
D = 128
G = 16
GP = 32

NSUB = 16
NPT = 640
NP = NSUB * NPT
CH = 64
ECH = 314
EPT = ECH * CH
EP = NSUB * EPT
NCH = NPT // CH


def _sc_passes(x0, x1, s0, d0, s1, d1, b0, b1, z128, ones128):
    mesh = plsc.VectorSubcoreMesh(core_axis_name="c", subcore_axis_name="s")
    f32 = jnp.float32
    outs = (
        jax.ShapeDtypeStruct((NP, D), f32),
        jax.ShapeDtypeStruct((NP, D), f32),
        jax.ShapeDtypeStruct((GP, D), f32),
        jax.ShapeDtypeStruct((GP, D), f32),
        jax.ShapeDtypeStruct((GP, D), f32),
        jax.ShapeDtypeStruct((GP, D), f32),
        jax.ShapeDtypeStruct((GP, D), f32),
        jax.ShapeDtypeStruct((GP, D), f32),
        jax.ShapeDtypeStruct((GP, D), f32),
        jax.ShapeDtypeStruct((GP, D), f32),
        jax.ShapeDtypeStruct((GP, D), f32),
        jax.ShapeDtypeStruct((GP, D), f32),
    )

    @functools.partial(
        pl.kernel,
        mesh=mesh,
        out_type=outs,
        scratch_types=[
            pltpu.VMEM((2, CH), jnp.int32),
            pltpu.VMEM((2, CH), jnp.int32),
            pltpu.VMEM((CH, D), f32),
            pltpu.VMEM((CH, D), f32),
            pltpu.VMEM((NPT // 8, D), f32),
            pltpu.VMEM((CH, D), f32),
            pltpu.VMEM((CH, D), f32),
            pltpu.VMEM((NCH, CH), jnp.int32),
            pltpu.VMEM_SHARED((NP, D), f32),
            pltpu.VMEM_SHARED((GP, D), f32),
            pltpu.VMEM_SHARED((GP, D), f32),
            pltpu.VMEM_SHARED((GP, D), f32),
            pltpu.VMEM_SHARED((GP, D), f32),
            pltpu.VMEM_SHARED((GP, D), f32),
            pltpu.SemaphoreType.DMA,
            pltpu.SemaphoreType.DMA,
        ],
    )
    def body(x0r, x1r, s0r, d0r, s1r, d1r, b0r, b1r, z128r, ones128r,
             y0r, y1r, px0r, py0r, pw0r, pm0r, nc0r,
             px1r, py1r, pw1r, pm1r, nc1r,
             srcv, dstv, rows, rowsB, cntv, mv, onesv, bv,
             acc, pxS, pyS, pwS, pmS, ncS, sem, semB):
        c = lax.axis_index("c")
        s = lax.axis_index("s")
        r0 = s * NPT

        def cnt_vec(r):
            return cntv[r // 8, pl.ds((r % 8) * 16, 16)]

        def divide_rows():
            def rowfix(i, carry):
                dv = jnp.maximum(cnt_vec(carry + i), 1.0)
                for cc in range(D // 16):
                    rows[i, pl.ds(cc * 16, 16)] = rows[i, pl.ds(cc * 16, 16)] / dv
                return carry
            return rowfix

        def zero_acc_slice():
            def zrow(k, carry):
                pltpu.sync_copy(rows, acc.at[pl.ds(r0 + k * CH, CH)])
                return carry
            lax.fori_loop(0, NCH, zrow, 0)

        pltpu.sync_copy(ones128r, onesv)
        pltpu.sync_copy(z128r, rows)
        zero_acc_slice()

        @pl.when(s == 0)
        def _():
            pltpu.sync_copy(rows.at[pl.ds(0, GP)], pxS)
            pltpu.sync_copy(rows.at[pl.ds(0, GP)], pyS)
            pltpu.sync_copy(rows.at[pl.ds(0, GP)], pwS)
            pltpu.sync_copy(rows.at[pl.ds(0, GP)], pmS)
            pltpu.sync_copy(rows.at[pl.ds(0, GP)], ncS)

        @pl.when(c == 0)
        def _():
            pltpu.sync_copy(b0r.at[s], bv)

        @pl.when(c == 1)
        def _():
            pltpu.sync_copy(b1r.at[s], bv)

        plsc.subcore_barrier()

        pltpu.sync_copy(ones128r, rows)

        def p0_loop(dr):
            pltpu.sync_copy(dr.at[s, 0], dstv.at[0])
            pltpu.async_copy(rows, acc.at[dstv.at[0]], sem, add=True)

            def pair(j2, carry):
                jB = 2 * j2 + 1
                pltpu.sync_copy(dr.at[s, jB], dstv.at[1])
                pltpu.async_copy(rows, acc.at[dstv.at[1]], semB, add=True)
                pltpu.make_async_copy(rows, acc.at[dstv.at[0]], sem).wait()

                @pl.when(j2 + 1 < ECH // 2)
                def _():
                    pltpu.sync_copy(dr.at[s, jB + 1], dstv.at[0])
                    pltpu.async_copy(rows, acc.at[dstv.at[0]], sem, add=True)

                pltpu.make_async_copy(rows, acc.at[dstv.at[1]], semB).wait()
                return carry
            lax.fori_loop(0, ECH // 2, pair, 0)

        @pl.when(c == 0)
        def _():
            p0_loop(d0r)

        @pl.when(c == 1)
        def _():
            p0_loop(d1r)

        plsc.subcore_barrier()

        def pext(k, carry):
            pltpu.sync_copy(acc.at[pl.ds(r0 + k * CH, CH)], rows)

            def crow(i, carry2):
                r = carry2 + i
                cntv[r // 8, pl.ds((r % 8) * 16, 16)] = rows[i, pl.ds(0, 16)]
                return carry2
            lax.fori_loop(0, CH, crow, k * CH)
            return carry
        lax.fori_loop(0, NCH, pext, 0)
        pltpu.sync_copy(z128r, rows)
        zero_acc_slice()
        plsc.subcore_barrier()

        def p1_loop(xr, sr, dr):
            pltpu.sync_copy(sr.at[s, 0], srcv.at[0])
            pltpu.sync_copy(dr.at[s, 0], dstv.at[0])
            pltpu.async_copy(xr.at[srcv.at[0]], rows, sem)

            def pair(j2, carry):
                jB = 2 * j2 + 1
                pltpu.sync_copy(sr.at[s, jB], srcv.at[1])
                pltpu.sync_copy(dr.at[s, jB], dstv.at[1])
                pltpu.async_copy(xr.at[srcv.at[1]], rowsB, semB)
                pltpu.make_async_copy(xr.at[srcv.at[0]], rows, sem).wait()
                pltpu.sync_copy(rows, acc.at[dstv.at[0]], add=True)

                @pl.when(j2 + 1 < ECH // 2)
                def _():
                    pltpu.sync_copy(sr.at[s, jB + 1], srcv.at[0])
                    pltpu.sync_copy(dr.at[s, jB + 1], dstv.at[0])
                    pltpu.async_copy(xr.at[srcv.at[0]], rows, sem)

                pltpu.make_async_copy(xr.at[srcv.at[1]], rowsB, semB).wait()
                pltpu.sync_copy(rowsB, acc.at[dstv.at[1]], add=True)
                return carry
            lax.fori_loop(0, ECH // 2, pair, 0)

        @pl.when(c == 0)
        def _():
            p1_loop(x0r, s0r, d0r)

        @pl.when(c == 1)
        def _():
            p1_loop(x1r, s1r, d1r)

        plsc.subcore_barrier()

        def p2_loop(xr, yr):
            def p2(k, carry):
                row0 = r0 + k * CH
                pltpu.sync_copy(acc.at[pl.ds(row0, CH)], rows)
                lax.fori_loop(0, CH, divide_rows(), k * CH)

                def mrow(i, carry2):
                    mval = jnp.where(cnt_vec(carry2 + i) > 0.0, 1.0, 0.0)
                    for cc in range(D // 16):
                        mv[i, pl.ds(cc * 16, 16)] = mval
                    return carry2
                lax.fori_loop(0, CH, mrow, k * CH)
                pltpu.sync_copy(rows, yr.at[pl.ds(row0, CH)])
                pltpu.sync_copy(rows, pyS.at[bv.at[k]], add=True)
                pltpu.sync_copy(mv, pmS.at[bv.at[k]], add=True)
                pltpu.sync_copy(onesv, ncS.at[bv.at[k]], add=True)
                pltpu.sync_copy(xr.at[pl.ds(row0, CH)], rows)
                pltpu.sync_copy(rows, pxS.at[bv.at[k]], add=True)
                return carry
            lax.fori_loop(0, NCH, p2, 0)

        @pl.when(c == 0)
        def _():
            p2_loop(x0r, y0r)

        @pl.when(c == 1)
        def _():
            p2_loop(x1r, y1r)

        pltpu.sync_copy(z128r, rows)
        zero_acc_slice()
        plsc.subcore_barrier()

        def p3_loop(sr, dr, yr):
            pltpu.sync_copy(sr.at[s, 0], srcv.at[0])
            pltpu.sync_copy(dr.at[s, 0], dstv.at[0])
            pltpu.async_copy(yr.at[srcv.at[0]], rows, sem)

            def pair(j2, carry):
                jB = 2 * j2 + 1
                pltpu.sync_copy(sr.at[s, jB], srcv.at[1])
                pltpu.sync_copy(dr.at[s, jB], dstv.at[1])
                pltpu.async_copy(yr.at[srcv.at[1]], rowsB, semB)
                pltpu.make_async_copy(yr.at[srcv.at[0]], rows, sem).wait()
                pltpu.sync_copy(rows, acc.at[dstv.at[0]], add=True)

                @pl.when(j2 + 1 < ECH // 2)
                def _():
                    pltpu.sync_copy(sr.at[s, jB + 1], srcv.at[0])
                    pltpu.sync_copy(dr.at[s, jB + 1], dstv.at[0])
                    pltpu.async_copy(yr.at[srcv.at[0]], rows, sem)

                pltpu.make_async_copy(yr.at[srcv.at[1]], rowsB, semB).wait()
                pltpu.sync_copy(rowsB, acc.at[dstv.at[1]], add=True)
                return carry
            lax.fori_loop(0, ECH // 2, pair, 0)

        @pl.when(c == 0)
        def _():
            p3_loop(s0r, d0r, y0r)

        @pl.when(c == 1)
        def _():
            p3_loop(s1r, d1r, y1r)

        plsc.subcore_barrier()

        def p4(k, carry):
            row0 = r0 + k * CH
            pltpu.sync_copy(acc.at[pl.ds(row0, CH)], rows)
            lax.fori_loop(0, CH, divide_rows(), k * CH)
            pltpu.sync_copy(rows, pwS.at[bv.at[k]], add=True)
            return carry
        lax.fori_loop(0, NCH, p4, 0)
        plsc.subcore_barrier()

        def publish(pxo, pyo, pwo, pmo, nco):
            pltpu.sync_copy(pxS, rows.at[pl.ds(0, GP)])
            pltpu.sync_copy(rows.at[pl.ds(0, GP)], pxo)
            pltpu.sync_copy(pyS, rows.at[pl.ds(0, GP)])
            pltpu.sync_copy(rows.at[pl.ds(0, GP)], pyo)
            pltpu.sync_copy(pwS, rows.at[pl.ds(0, GP)])
            pltpu.sync_copy(rows.at[pl.ds(0, GP)], pwo)
            pltpu.sync_copy(pmS, rows.at[pl.ds(0, GP)])
            pltpu.sync_copy(rows.at[pl.ds(0, GP)], pmo)
            pltpu.sync_copy(ncS, rows.at[pl.ds(0, GP)])
            pltpu.sync_copy(rows.at[pl.ds(0, GP)], nco)

        @pl.when((s == 0) & (c == 0))
        def _():
            publish(px0r, py0r, pw0r, pm0r, nc0r)

        @pl.when((s == 0) & (c == 1))
        def _():
            publish(px1r, py1r, pw1r, pm1r, nc1r)

    return body(x0, x1, s0, d0, s1, d1, b0, b1, z128, ones128)


def _mm(a, b):
    return lax.dot_general(a, b, (((1,), (1,)), ((), ())),
                           precision=lax.Precision.HIGHEST,
                           preferred_element_type=jnp.float32)


def _sigmoid(x):
    return 1.0 / (1.0 + jnp.exp(-x))


def _tc_dense(px0, py0, pw0, pm0, nc0, px1, py1, pw1, pm1, nc1,
              Wl1, bl1, Wr1, Wl2, bl2, Wr2, Wc1, bc1, Wc2, bc2):
    def body(px0r, py0r, pw0r, pm0r, nc0r, px1r, py1r, pw1r, pm1r, nc1r,
             Wl1r, bl1r, Wr1r, Wl2r, bl2r, Wr2r, Wc1r, bc1r, Wc2r, bc2r,
             outr):
        A1, B1 = Wl1r[...], bl1r[...]
        R1 = Wr1r[...]
        A2, B2, R2 = Wl2r[...], bl2r[...], Wr2r[...]

        def graph(pxr, pyr, pwr, pmr, ncr):
            nc = ncr[...][:G, 0:1]
            inv = 1.0 / jnp.maximum(nc, 1.0)
            u = jnp.where(nc > 0.0, 1.0, 0.0)
            px = pxr[...][:G, :] * inv
            py = pyr[...][:G, :] * inv
            pw = pwr[...][:G, :] * inv
            pm = pmr[...][:G, 0:1] * inv
            Pf = _mm(py, A1) + _mm(px, R1) + u * B1
            Pz = _mm(pw, A1) + _mm(py, R1) + pm * B1
            return _mm(Pz, A2) + _mm(Pf, R2) + u * B2

        e0 = graph(px0r, py0r, pw0r, pm0r, nc0r)
        e1 = graph(px1r, py1r, pw1r, pm1r, nc1r)
        comb = jnp.concatenate([e0, e1], axis=1)
        h = _sigmoid(_mm(comb, Wc1r[...]) + bc1r[...])
        o = jnp.sum(h * Wc2r[...], axis=1, keepdims=True) + bc2r[...][0, 0]
        outr[...] = _sigmoid(o)

    return pl.pallas_call(
        body,
        out_shape=jax.ShapeDtypeStruct((G, 1), jnp.float32),
    )(px0, py0, pw0, pm0, nc0, px1, py1, pw1, pm1, nc1,
      Wl1, bl1.reshape(1, -1), Wr1, Wl2, bl2.reshape(1, -1), Wr2,
      Wc1, bc1.reshape(1, -1), Wc2, bc2.reshape(1, 1))


def _prep_graph(x, edge_index, batch):
    src = edge_index[0].astype(jnp.int32)
    dst = edge_index[1].astype(jnp.int32)
    srcp = jnp.concatenate([src, jnp.zeros((EP - E,), jnp.int32)]).reshape(NSUB, ECH, CH)
    dstp = jnp.concatenate([dst, jnp.full((EP - E,), N, jnp.int32)]).reshape(NSUB, ECH, CH)
    xp = jnp.concatenate([x, jnp.zeros((NP - N, D), x.dtype)], axis=0)
    bp = jnp.concatenate([batch.astype(jnp.int32),
                          jnp.full((NP - N,), G, jnp.int32)]).reshape(NSUB, NCH, CH)
    return xp, srcp, dstp, bp


def kernel(x0, edge_index0, batch0, x1, edge_index1, batch1,
           Wl1, bl1, Wr1, Wl2, bl2, Wr2, Wc1, bc1, Wc2, bc2):
    x0p, s0, d0, b0 = _prep_graph(x0, edge_index0, batch0)
    x1p, s1, d1, b1 = _prep_graph(x1, edge_index1, batch1)
    z128 = jnp.zeros((CH, D), jnp.float32)
    ones128 = jnp.ones((CH, D), jnp.float32)

    hbm = lambda a: pltpu.with_memory_space_constraint(a, pltpu.HBM)
    (_, _, px0, py0, pw0, pm0, nc0, px1, py1, pw1, pm1, nc1) = _sc_passes(
        hbm(x0p), hbm(x1p), hbm(s0), hbm(d0), hbm(s1), hbm(d1),
        hbm(b0), hbm(b1), hbm(z128), hbm(ones128))

    prob = _tc_dense(px0, py0, pw0, pm0, nc0, px1, py1, pw1, pm1, nc1,
                     Wl1, bl1, Wr1, Wl2, bl2, Wr2, Wc1, bc1, Wc2, bc2)
    return jnp.squeeze(prob, axis=-1)


# speedup vs baseline: 5.7412x; 1.0662x over previous
import functools

import jax
import jax.numpy as jnp
from jax import lax
from jax.experimental import pallas as pl
from jax.experimental.pallas import tpu as pltpu
from jax.experimental.pallas import tpu_sc as plsc

N = 10000
E = 320000
D = 128
G = 16
GP = 32

NSUB = 16
NPT = 640
NP = NSUB * NPT
CH = 64
ECH = 314
EPT = ECH * CH
EP = NSUB * EPT
NCH = NPT // CH
LR = NP // 128
LRT = NPT // 128


def _sc_passes(x0, x1, s0, d0, s1, d1, b0, b1, z128, ones128):
    mesh = plsc.VectorSubcoreMesh(core_axis_name="c", subcore_axis_name="s")
    f32 = jnp.float32
    outs = (
        jax.ShapeDtypeStruct((NP, D), f32),
        jax.ShapeDtypeStruct((NP, D), f32),
        jax.ShapeDtypeStruct((GP, D), f32),
        jax.ShapeDtypeStruct((GP, D), f32),
        jax.ShapeDtypeStruct((GP, D), f32),
        jax.ShapeDtypeStruct((GP, D), f32),
        jax.ShapeDtypeStruct((GP, D), f32),
        jax.ShapeDtypeStruct((GP, D), f32),
        jax.ShapeDtypeStruct((GP, D), f32),
        jax.ShapeDtypeStruct((GP, D), f32),
        jax.ShapeDtypeStruct((GP, D), f32),
        jax.ShapeDtypeStruct((GP, D), f32),
    )

    @functools.partial(
        pl.kernel,
        mesh=mesh,
        compiler_params=pltpu.CompilerParams(needs_layout_passes=False),
        out_type=outs,
        scratch_types=[
            pltpu.VMEM((2, CH), jnp.int32),
            pltpu.VMEM((2, CH), jnp.int32),
            pltpu.VMEM((CH, D), f32),
            pltpu.VMEM((CH, D), f32),
            pltpu.VMEM((LR, D), f32),
            pltpu.VMEM((LRT, D), f32),
            pltpu.VMEM((NCH, CH), jnp.int32),
            pltpu.VMEM_SHARED((NP, D), f32),
            pltpu.VMEM_SHARED((NSUB, LR, D), f32),
            pltpu.VMEM_SHARED((GP, D), f32),
            pltpu.VMEM_SHARED((GP, D), f32),
            pltpu.VMEM_SHARED((GP, D), f32),
            pltpu.VMEM_SHARED((GP, D), f32),
            pltpu.VMEM_SHARED((GP, D), f32),
            pltpu.SemaphoreType.DMA,
            pltpu.SemaphoreType.DMA,
        ],
    )
    def body(x0r, x1r, s0r, d0r, s1r, d1r, b0r, b1r, z128r, ones128r,
             y0r, y1r, px0r, py0r, pw0r, pm0r, nc0r,
             px1r, py1r, pw1r, pm1r, nc1r,
             srcv, dstv, rows, rowsB, loc, cntf, bv,
             acc, cntW, pxS, pyS, pwS, pmS, ncS, sem, semB):
        c = lax.axis_index("c")
        s = lax.axis_index("s")
        r0 = s * NPT
        i32 = jnp.int32

        def cnt_vec(r):
            ii = jnp.zeros((16,), i32) + r
            return plsc.load_gather(
                cntf, [lax.shift_right_logical(ii, 7), lax.bitwise_and(ii, 127)])

        def divide_rows():
            def rowfix(i, carry):
                dv = jnp.maximum(cnt_vec(carry + i), 1.0)
                for cc in range(D // 16):
                    rows[i, pl.ds(cc * 16, 16)] = rows[i, pl.ds(cc * 16, 16)] / dv
                return carry
            return rowfix

        def zero_acc_slice():
            def zrow(k, carry):
                pltpu.sync_copy(rows, acc.at[pl.ds(r0 + k * CH, CH)])
                return carry
            lax.fori_loop(0, NCH, zrow, 0)

        pltpu.sync_copy(z128r, rows)
        zero_acc_slice()
        pltpu.sync_copy(z128r, loc.at[pl.ds(0, CH)])
        pltpu.sync_copy(z128r.at[pl.ds(0, LR - CH)], loc.at[pl.ds(CH, LR - CH)])
        pltpu.sync_copy(z128r.at[pl.ds(0, LRT)], cntf)

        @pl.when(s == 0)
        def _():
            pltpu.sync_copy(rows.at[pl.ds(0, GP)], pxS)
            pltpu.sync_copy(rows.at[pl.ds(0, GP)], pyS)
            pltpu.sync_copy(rows.at[pl.ds(0, GP)], pwS)
            pltpu.sync_copy(rows.at[pl.ds(0, GP)], pmS)
            pltpu.sync_copy(rows.at[pl.ds(0, GP)], ncS)

        @pl.when(c == 0)
        def _():
            pltpu.sync_copy(b0r.at[s], bv)

        @pl.when(c == 1)
        def _():
            pltpu.sync_copy(b1r.at[s], bv)

        ones16 = jnp.ones((16,), f32)

        def hist_slot(slot):
            for u in range(CH // 16):
                d = dstv[slot, pl.ds(u * 16, 16)]
                plsc.addupdate_scatter(
                    loc, [lax.shift_right_logical(d, 7),
                          lax.bitwise_and(d, 127)], ones16)

        def p0_loop(dr):
            pltpu.async_copy(dr.at[s, 0], dstv.at[0], sem)

            def pair(j2, carry):
                jB = 2 * j2 + 1
                pltpu.async_copy(dr.at[s, jB], dstv.at[1], semB)
                pltpu.make_async_copy(dr.at[s, 0], dstv.at[0], sem).wait()
                hist_slot(0)

                @pl.when(j2 + 1 < ECH // 2)
                def _():
                    pltpu.async_copy(dr.at[s, jB + 1], dstv.at[0], sem)

                pltpu.make_async_copy(dr.at[s, 0], dstv.at[1], semB).wait()
                hist_slot(1)
                return carry
            lax.fori_loop(0, ECH // 2, pair, 0)

        @pl.when(c == 0)
        def _():
            p0_loop(d0r)

        @pl.when(c == 1)
        def _():
            p0_loop(d1r)

        pltpu.sync_copy(loc, cntW.at[s])
        plsc.subcore_barrier()

        def mrg(t, carry):
            pltpu.sync_copy(cntW.at[t, pl.ds(LRT * s, LRT)], rows.at[pl.ds(0, LRT)])
            for r in range(LRT):
                for cc in range(D // 16):
                    cntf[r, pl.ds(cc * 16, 16)] = (
                        cntf[r, pl.ds(cc * 16, 16)] + rows[r, pl.ds(cc * 16, 16)])
            return carry
        lax.fori_loop(0, NSUB, mrg, 0)
        pltpu.sync_copy(ones128r, loc.at[pl.ds(0, CH)])

        def gather_scatter_loop(tab, sr, dr):
            pltpu.sync_copy(sr.at[s, 0], srcv.at[0])
            pltpu.sync_copy(dr.at[s, 0], dstv.at[0])
            pltpu.async_copy(tab.at[srcv.at[0]], rows, sem)

            def pair(j2, carry):
                jB = 2 * j2 + 1
                pltpu.sync_copy(sr.at[s, jB], srcv.at[1])
                pltpu.sync_copy(dr.at[s, jB], dstv.at[1])
                pltpu.async_copy(tab.at[srcv.at[1]], rowsB, semB)
                pltpu.make_async_copy(tab.at[srcv.at[0]], rows, sem).wait()
                pltpu.sync_copy(rows, acc.at[dstv.at[0]], add=True)

                @pl.when(j2 + 1 < ECH // 2)
                def _():
                    pltpu.sync_copy(sr.at[s, jB + 1], srcv.at[0])
                    pltpu.sync_copy(dr.at[s, jB + 1], dstv.at[0])
                    pltpu.async_copy(tab.at[srcv.at[0]], rows, sem)

                pltpu.make_async_copy(tab.at[srcv.at[1]], rowsB, semB).wait()
                pltpu.sync_copy(rowsB, acc.at[dstv.at[1]], add=True)
                return carry
            lax.fori_loop(0, ECH // 2, pair, 0)

        @pl.when(c == 0)
        def _():
            gather_scatter_loop(x0r, s0r, d0r)

        @pl.when(c == 1)
        def _():
            gather_scatter_loop(x1r, s1r, d1r)

        plsc.subcore_barrier()

        def p2_loop(xr, yr):
            def p2(k, carry):
                row0 = r0 + k * CH
                pltpu.sync_copy(acc.at[pl.ds(row0, CH)], rows)
                lax.fori_loop(0, CH, divide_rows(), k * CH)

                def mrow(i, carry2):
                    mval = jnp.where(cnt_vec(carry2 + i) > 0.0, 1.0, 0.0)
                    for cc in range(D // 16):
                        rowsB[i, pl.ds(cc * 16, 16)] = mval
                    return carry2
                lax.fori_loop(0, CH, mrow, k * CH)
                pltpu.sync_copy(rows, yr.at[pl.ds(row0, CH)])
                pltpu.sync_copy(rows, pyS.at[bv.at[k]], add=True)
                pltpu.sync_copy(rowsB, pmS.at[bv.at[k]], add=True)
                pltpu.sync_copy(loc.at[pl.ds(0, CH)], ncS.at[bv.at[k]], add=True)
                pltpu.sync_copy(xr.at[pl.ds(row0, CH)], rows)
                pltpu.sync_copy(rows, pxS.at[bv.at[k]], add=True)
                return carry
            lax.fori_loop(0, NCH, p2, 0)

        @pl.when(c == 0)
        def _():
            p2_loop(x0r, y0r)

        @pl.when(c == 1)
        def _():
            p2_loop(x1r, y1r)

        pltpu.sync_copy(z128r, rows)
        zero_acc_slice()
        plsc.subcore_barrier()

        @pl.when(c == 0)
        def _():
            gather_scatter_loop(y0r, s0r, d0r)

        @pl.when(c == 1)
        def _():
            gather_scatter_loop(y1r, s1r, d1r)

        plsc.subcore_barrier()

        def p4(k, carry):
            row0 = r0 + k * CH
            pltpu.sync_copy(acc.at[pl.ds(row0, CH)], rows)
            lax.fori_loop(0, CH, divide_rows(), k * CH)
            pltpu.sync_copy(rows, pwS.at[bv.at[k]], add=True)
            return carry
        lax.fori_loop(0, NCH, p4, 0)
        plsc.subcore_barrier()

        def publish(pxo, pyo, pwo, pmo, nco):
            pltpu.sync_copy(pxS, rows.at[pl.ds(0, GP)])
            pltpu.sync_copy(rows.at[pl.ds(0, GP)], pxo)
            pltpu.sync_copy(pyS, rows.at[pl.ds(0, GP)])
            pltpu.sync_copy(rows.at[pl.ds(0, GP)], pyo)
            pltpu.sync_copy(pwS, rows.at[pl.ds(0, GP)])
            pltpu.sync_copy(rows.at[pl.ds(0, GP)], pwo)
            pltpu.sync_copy(pmS, rows.at[pl.ds(0, GP)])
            pltpu.sync_copy(rows.at[pl.ds(0, GP)], pmo)
            pltpu.sync_copy(ncS, rows.at[pl.ds(0, GP)])
            pltpu.sync_copy(rows.at[pl.ds(0, GP)], nco)

        @pl.when((s == 0) & (c == 0))
        def _():
            publish(px0r, py0r, pw0r, pm0r, nc0r)

        @pl.when((s == 0) & (c == 1))
        def _():
            publish(px1r, py1r, pw1r, pm1r, nc1r)

    return body(x0, x1, s0, d0, s1, d1, b0, b1, z128, ones128)


def _mm(a, b):
    return lax.dot_general(a, b, (((1,), (1,)), ((), ())),
                           precision=lax.Precision.HIGHEST,
                           preferred_element_type=jnp.float32)


def _sigmoid(x):
    return 1.0 / (1.0 + jnp.exp(-x))


def _tc_dense(px0, py0, pw0, pm0, nc0, px1, py1, pw1, pm1, nc1,
              Wl1, bl1, Wr1, Wl2, bl2, Wr2, Wc1, bc1, Wc2, bc2):
    def body(px0r, py0r, pw0r, pm0r, nc0r, px1r, py1r, pw1r, pm1r, nc1r,
             Wl1r, bl1r, Wr1r, Wl2r, bl2r, Wr2r, Wc1r, bc1r, Wc2r, bc2r,
             outr):
        A1, B1 = Wl1r[...], bl1r[...]
        R1 = Wr1r[...]
        A2, B2, R2 = Wl2r[...], bl2r[...], Wr2r[...]

        def graph(pxr, pyr, pwr, pmr, ncr):
            nc = ncr[...][:G, 0:1]
            inv = 1.0 / jnp.maximum(nc, 1.0)
            u = jnp.where(nc > 0.0, 1.0, 0.0)
            px = pxr[...][:G, :] * inv
            py = pyr[...][:G, :] * inv
            pw = pwr[...][:G, :] * inv
            pm = pmr[...][:G, 0:1] * inv
            Pf = _mm(py, A1) + _mm(px, R1) + u * B1
            Pz = _mm(pw, A1) + _mm(py, R1) + pm * B1
            return _mm(Pz, A2) + _mm(Pf, R2) + u * B2

        e0 = graph(px0r, py0r, pw0r, pm0r, nc0r)
        e1 = graph(px1r, py1r, pw1r, pm1r, nc1r)
        comb = jnp.concatenate([e0, e1], axis=1)
        h = _sigmoid(_mm(comb, Wc1r[...]) + bc1r[...])
        o = jnp.sum(h * Wc2r[...], axis=1, keepdims=True) + bc2r[...][0, 0]
        outr[...] = _sigmoid(o)

    return pl.pallas_call(
        body,
        out_shape=jax.ShapeDtypeStruct((G, 1), jnp.float32),
    )(px0, py0, pw0, pm0, nc0, px1, py1, pw1, pm1, nc1,
      Wl1, bl1.reshape(1, -1), Wr1, Wl2, bl2.reshape(1, -1), Wr2,
      Wc1, bc1.reshape(1, -1), Wc2, bc2.reshape(1, 1))


def _prep_graph(x, edge_index, batch):
    src = edge_index[0].astype(jnp.int32)
    dst = edge_index[1].astype(jnp.int32)
    srcp = jnp.concatenate([src, jnp.zeros((EP - E,), jnp.int32)]).reshape(NSUB, ECH, CH)
    dstp = jnp.concatenate([dst, jnp.full((EP - E,), N, jnp.int32)]).reshape(NSUB, ECH, CH)
    xp = jnp.concatenate([x, jnp.zeros((NP - N, D), x.dtype)], axis=0)
    bp = jnp.concatenate([batch.astype(jnp.int32),
                          jnp.full((NP - N,), G, jnp.int32)]).reshape(NSUB, NCH, CH)
    return xp, srcp, dstp, bp


def kernel(x0, edge_index0, batch0, x1, edge_index1, batch1,
           Wl1, bl1, Wr1, Wl2, bl2, Wr2, Wc1, bc1, Wc2, bc2):
    x0p, s0, d0, b0 = _prep_graph(x0, edge_index0, batch0)
    x1p, s1, d1, b1 = _prep_graph(x1, edge_index1, batch1)
    z128 = jnp.zeros((CH, D), jnp.float32)
    ones128 = jnp.ones((CH, D), jnp.float32)

    hbm = lambda a: pltpu.with_memory_space_constraint(a, pltpu.HBM)
    (_, _, px0, py0, pw0, pm0, nc0, px1, py1, pw1, pm1, nc1) = _sc_passes(
        hbm(x0p), hbm(x1p), hbm(s0), hbm(d0), hbm(s1), hbm(d1),
        hbm(b0), hbm(b1), hbm(z128), hbm(ones128))

    prob = _tc_dense(px0, py0, pw0, pm0, nc0, px1, py1, pw1, pm1, nc1,
                     Wl1, bl1, Wr1, Wl2, bl2, Wr2, Wc1, bc1, Wc2, bc2)
    return jnp.squeeze(prob, axis=-1)


# speedup vs baseline: 6.2941x; 1.0963x over previous
import functools

import jax
import jax.numpy as jnp
from jax import lax
from jax.experimental import pallas as pl
from jax.experimental.pallas import tpu as pltpu
from jax.experimental.pallas import tpu_sc as plsc

N = 10000
E = 320000
D = 128
G = 16
GP = 32

NSUB = 16
NPT = 640
NP = NSUB * NPT
CH = 128
ECH = 158
EPT = ECH * CH
EP = NSUB * EPT
NCH = NPT // CH
NCH64 = NPT // 64
LR = NP // 128
LRT = NPT // 128


def _sc_passes(x0, x1, s0, d0, s1, d1, b0, b1, z128, ones128):
    mesh = plsc.VectorSubcoreMesh(core_axis_name="c", subcore_axis_name="s")
    f32 = jnp.float32
    outs = (
        jax.ShapeDtypeStruct((NP, D), f32),
        jax.ShapeDtypeStruct((NP, D), f32),
        jax.ShapeDtypeStruct((GP, D), f32),
        jax.ShapeDtypeStruct((GP, D), f32),
        jax.ShapeDtypeStruct((GP, D), f32),
        jax.ShapeDtypeStruct((GP, D), f32),
        jax.ShapeDtypeStruct((GP, D), f32),
        jax.ShapeDtypeStruct((GP, D), f32),
        jax.ShapeDtypeStruct((GP, D), f32),
        jax.ShapeDtypeStruct((GP, D), f32),
        jax.ShapeDtypeStruct((GP, D), f32),
        jax.ShapeDtypeStruct((GP, D), f32),
    )

    @functools.partial(
        pl.kernel,
        mesh=mesh,
        compiler_params=pltpu.CompilerParams(needs_layout_passes=False),
        out_type=outs,
        scratch_types=[
            pltpu.VMEM((2, CH), jnp.int32),
            pltpu.VMEM((2, CH), jnp.int32),
            pltpu.VMEM((CH, D), f32),
            pltpu.VMEM((CH, D), f32),
            pltpu.VMEM((LR, D), f32),
            pltpu.VMEM((LRT, D), f32),
            pltpu.VMEM((NCH, CH), jnp.int32),
            pltpu.VMEM((1, 64), jnp.int32),
            pltpu.VMEM((1, 16), jnp.int32),
            pltpu.VMEM_SHARED((NP, D), f32),
            pltpu.VMEM_SHARED((LR, D), f32),
            pltpu.VMEM_SHARED((GP, D), f32),
            pltpu.VMEM_SHARED((GP, D), f32),
            pltpu.VMEM_SHARED((GP, D), f32),
            pltpu.VMEM_SHARED((GP, D), f32),
            pltpu.VMEM_SHARED((GP, D), f32),
            pltpu.SemaphoreType.DMA,
            pltpu.SemaphoreType.DMA,
        ],
    )
    def body(x0r, x1r, s0r, d0r, s1r, d1r, b0r, b1r, z128r, ones128r,
             y0r, y1r, px0r, py0r, pw0r, pm0r, nc0r,
             px1r, py1r, pw1r, pm1r, nc1r,
             srcv, dstv, rows, rowsB, loc, cntf, bv, idxA, idxB,
             acc, cntW, pxS, pyS, pwS, pmS, ncS, sem, semB):
        c = lax.axis_index("c")
        s = lax.axis_index("s")
        r0 = s * NPT
        i32 = jnp.int32

        def cnt_vec(r):
            ii = jnp.zeros((16,), i32) + r
            return plsc.load_gather(
                cntf, [lax.shift_right_logical(ii, 7), lax.bitwise_and(ii, 127)])

        def divide_rows():
            def rowfix(i, carry):
                dv = jnp.maximum(cnt_vec(carry + i), 1.0)
                for cc in range(D // 16):
                    rows[i, pl.ds(cc * 16, 16)] = rows[i, pl.ds(cc * 16, 16)] / dv
                return carry
            return rowfix

        def zero_acc_slice():
            def zrow(k, carry):
                pltpu.sync_copy(rows, acc.at[pl.ds(r0 + k * CH, CH)])
                return carry
            lax.fori_loop(0, NCH, zrow, 0)

        pltpu.sync_copy(z128r, rows)
        zero_acc_slice()
        pltpu.sync_copy(z128r.at[pl.ds(0, LR)], loc)
        pltpu.sync_copy(z128r.at[pl.ds(0, LRT)], cntf)
        for u in range(4):
            idxA[0, pl.ds(u * 16, 16)] = lax.iota(i32, 16) + (u * 16)
        idxB[0, :] = lax.iota(i32, 16) + 64

        @pl.when(s == 0)
        def _():
            pltpu.sync_copy(rows.at[pl.ds(0, GP)], pxS)
            pltpu.sync_copy(rows.at[pl.ds(0, GP)], pyS)
            pltpu.sync_copy(rows.at[pl.ds(0, GP)], pwS)
            pltpu.sync_copy(rows.at[pl.ds(0, GP)], pmS)
            pltpu.sync_copy(rows.at[pl.ds(0, GP)], ncS)
            pltpu.sync_copy(rows.at[pl.ds(0, LR)], cntW)

        @pl.when(c == 0)
        def _():
            pltpu.sync_copy(b0r.at[s], bv)

        @pl.when(c == 1)
        def _():
            pltpu.sync_copy(b1r.at[s], bv)

        plsc.subcore_barrier()

        ones16 = jnp.ones((16,), f32)

        def hist_slot(slot):
            for u in range(CH // 16):
                d = dstv[slot, pl.ds(u * 16, 16)]
                plsc.addupdate_scatter(
                    loc, [lax.shift_right_logical(d, 7),
                          lax.bitwise_and(d, 127)], ones16)

        def p0_loop(dr):
            pltpu.async_copy(dr.at[s, 0], dstv.at[0], sem)

            def pair(j2, carry):
                jB = 2 * j2 + 1
                pltpu.async_copy(dr.at[s, jB], dstv.at[1], semB)
                pltpu.make_async_copy(dr.at[s, 0], dstv.at[0], sem).wait()
                hist_slot(0)

                @pl.when(j2 + 1 < ECH // 2)
                def _():
                    pltpu.async_copy(dr.at[s, jB + 1], dstv.at[0], sem)

                pltpu.make_async_copy(dr.at[s, 0], dstv.at[1], semB).wait()
                hist_slot(1)
                return carry
            lax.fori_loop(0, ECH // 2, pair, 0)

        @pl.when(c == 0)
        def _():
            p0_loop(d0r)

        @pl.when(c == 1)
        def _():
            p0_loop(d1r)

        pltpu.sync_copy(loc.at[pl.ds(0, 64)], cntW.at[idxA.at[0]], add=True)
        pltpu.sync_copy(loc.at[pl.ds(64, 16)], cntW.at[idxB.at[0]], add=True)
        plsc.subcore_barrier()
        pltpu.sync_copy(cntW.at[pl.ds(LRT * s, LRT)], cntf)

        def gather_scatter_loop(tab, sr, dr):
            pltpu.sync_copy(sr.at[s, 0], srcv.at[0])
            pltpu.sync_copy(dr.at[s, 0], dstv.at[0])
            pltpu.async_copy(tab.at[srcv.at[0]], rows, sem)

            def pair(j2, carry):
                jB = 2 * j2 + 1
                pltpu.sync_copy(sr.at[s, jB], srcv.at[1])
                pltpu.sync_copy(dr.at[s, jB], dstv.at[1])
                pltpu.async_copy(tab.at[srcv.at[1]], rowsB, semB)
                pltpu.make_async_copy(tab.at[srcv.at[0]], rows, sem).wait()
                pltpu.sync_copy(rows, acc.at[dstv.at[0]], add=True)

                @pl.when(j2 + 1 < ECH // 2)
                def _():
                    pltpu.sync_copy(sr.at[s, jB + 1], srcv.at[0])
                    pltpu.sync_copy(dr.at[s, jB + 1], dstv.at[0])
                    pltpu.async_copy(tab.at[srcv.at[0]], rows, sem)

                pltpu.make_async_copy(tab.at[srcv.at[1]], rowsB, semB).wait()
                pltpu.sync_copy(rowsB, acc.at[dstv.at[1]], add=True)
                return carry
            lax.fori_loop(0, ECH // 2, pair, 0)

        @pl.when(c == 0)
        def _():
            gather_scatter_loop(x0r, s0r, d0r)

        @pl.when(c == 1)
        def _():
            gather_scatter_loop(x1r, s1r, d1r)

        plsc.subcore_barrier()

        def p2_loop(xr, yr):
            def p2(k, carry):
                row0 = r0 + k * CH
                pltpu.sync_copy(acc.at[pl.ds(row0, CH)], rows)
                lax.fori_loop(0, CH, divide_rows(), k * CH)

                def mrow(i, carry2):
                    mval = jnp.where(cnt_vec(carry2 + i) > 0.0, 1.0, 0.0)
                    for cc in range(D // 16):
                        rowsB[i, pl.ds(cc * 16, 16)] = mval
                    return carry2
                lax.fori_loop(0, CH, mrow, k * CH)
                pltpu.sync_copy(rows, yr.at[pl.ds(row0, CH)])
                pltpu.sync_copy(rows, pyS.at[bv.at[k]], add=True)
                pltpu.sync_copy(rowsB, pmS.at[bv.at[k]], add=True)
                pltpu.sync_copy(xr.at[pl.ds(row0, CH)], rows)
                pltpu.sync_copy(rows, pxS.at[bv.at[k]], add=True)
                pltpu.sync_copy(ones128r, rows)
                pltpu.sync_copy(rows, ncS.at[bv.at[k]], add=True)
                return carry
            lax.fori_loop(0, NCH, p2, 0)

        @pl.when(c == 0)
        def _():
            p2_loop(x0r, y0r)

        @pl.when(c == 1)
        def _():
            p2_loop(x1r, y1r)

        pltpu.sync_copy(z128r, rows)
        zero_acc_slice()
        plsc.subcore_barrier()

        @pl.when(c == 0)
        def _():
            gather_scatter_loop(y0r, s0r, d0r)

        @pl.when(c == 1)
        def _():
            gather_scatter_loop(y1r, s1r, d1r)

        plsc.subcore_barrier()

        def p4(k, carry):
            row0 = r0 + k * CH
            pltpu.sync_copy(acc.at[pl.ds(row0, CH)], rows)
            lax.fori_loop(0, CH, divide_rows(), k * CH)
            pltpu.sync_copy(rows, pwS.at[bv.at[k]], add=True)
            return carry
        lax.fori_loop(0, NCH, p4, 0)
        plsc.subcore_barrier()

        def publish(pxo, pyo, pwo, pmo, nco):
            pltpu.sync_copy(pxS, rows.at[pl.ds(0, GP)])
            pltpu.sync_copy(rows.at[pl.ds(0, GP)], pxo)
            pltpu.sync_copy(pyS, rows.at[pl.ds(0, GP)])
            pltpu.sync_copy(rows.at[pl.ds(0, GP)], pyo)
            pltpu.sync_copy(pwS, rows.at[pl.ds(0, GP)])
            pltpu.sync_copy(rows.at[pl.ds(0, GP)], pwo)
            pltpu.sync_copy(pmS, rows.at[pl.ds(0, GP)])
            pltpu.sync_copy(rows.at[pl.ds(0, GP)], pmo)
            pltpu.sync_copy(ncS, rows.at[pl.ds(0, GP)])
            pltpu.sync_copy(rows.at[pl.ds(0, GP)], nco)

        @pl.when((s == 0) & (c == 0))
        def _():
            publish(px0r, py0r, pw0r, pm0r, nc0r)

        @pl.when((s == 0) & (c == 1))
        def _():
            publish(px1r, py1r, pw1r, pm1r, nc1r)

    return body(x0, x1, s0, d0, s1, d1, b0, b1, z128, ones128)


def _mm(a, b):
    return lax.dot_general(a, b, (((1,), (1,)), ((), ())),
                           precision=lax.Precision.HIGHEST,
                           preferred_element_type=jnp.float32)


def _sigmoid(x):
    return 1.0 / (1.0 + jnp.exp(-x))


def _tc_dense(px0, py0, pw0, pm0, nc0, px1, py1, pw1, pm1, nc1,
              Wl1, bl1, Wr1, Wl2, bl2, Wr2, Wc1, bc1, Wc2, bc2):
    def body(px0r, py0r, pw0r, pm0r, nc0r, px1r, py1r, pw1r, pm1r, nc1r,
             Wl1r, bl1r, Wr1r, Wl2r, bl2r, Wr2r, Wc1r, bc1r, Wc2r, bc2r,
             outr):
        A1, B1 = Wl1r[...], bl1r[...]
        R1 = Wr1r[...]
        A2, B2, R2 = Wl2r[...], bl2r[...], Wr2r[...]

        def graph(pxr, pyr, pwr, pmr, ncr):
            nc = ncr[...][:G, 0:1]
            inv = 1.0 / jnp.maximum(nc, 1.0)
            u = jnp.where(nc > 0.0, 1.0, 0.0)
            px = pxr[...][:G, :] * inv
            py = pyr[...][:G, :] * inv
            pw = pwr[...][:G, :] * inv
            pm = pmr[...][:G, 0:1] * inv
            Pf = _mm(py, A1) + _mm(px, R1) + u * B1
            Pz = _mm(pw, A1) + _mm(py, R1) + pm * B1
            return _mm(Pz, A2) + _mm(Pf, R2) + u * B2

        e0 = graph(px0r, py0r, pw0r, pm0r, nc0r)
        e1 = graph(px1r, py1r, pw1r, pm1r, nc1r)
        comb = jnp.concatenate([e0, e1], axis=1)
        h = _sigmoid(_mm(comb, Wc1r[...]) + bc1r[...])
        o = jnp.sum(h * Wc2r[...], axis=1, keepdims=True) + bc2r[...][0, 0]
        outr[...] = _sigmoid(o)

    return pl.pallas_call(
        body,
        out_shape=jax.ShapeDtypeStruct((G, 1), jnp.float32),
    )(px0, py0, pw0, pm0, nc0, px1, py1, pw1, pm1, nc1,
      Wl1, bl1.reshape(1, -1), Wr1, Wl2, bl2.reshape(1, -1), Wr2,
      Wc1, bc1.reshape(1, -1), Wc2, bc2.reshape(1, 1))


def _prep_graph(x, edge_index, batch):
    src = edge_index[0].astype(jnp.int32)
    dst = edge_index[1].astype(jnp.int32)
    srcp = jnp.concatenate([src, jnp.zeros((EP - E,), jnp.int32)]).reshape(NSUB, ECH, CH)
    dstp = jnp.concatenate([dst, jnp.full((EP - E,), N, jnp.int32)]).reshape(NSUB, ECH, CH)
    xp = jnp.concatenate([x, jnp.zeros((NP - N, D), x.dtype)], axis=0)
    bflat = jnp.concatenate([batch.astype(jnp.int32),
                             jnp.full((NP - N,), G, jnp.int32)])
    bp = bflat.reshape(NSUB, NCH, CH)
    return xp, srcp, dstp, bp


def kernel(x0, edge_index0, batch0, x1, edge_index1, batch1,
           Wl1, bl1, Wr1, Wl2, bl2, Wr2, Wc1, bc1, Wc2, bc2):
    x0p, s0, d0, b0 = _prep_graph(x0, edge_index0, batch0)
    x1p, s1, d1, b1 = _prep_graph(x1, edge_index1, batch1)
    z128 = jnp.zeros((CH, D), jnp.float32)
    ones128 = jnp.ones((CH, D), jnp.float32)

    hbm = lambda a: pltpu.with_memory_space_constraint(a, pltpu.HBM)
    (_, _, px0, py0, pw0, pm0, nc0, px1, py1, pw1, pm1, nc1) = _sc_passes(
        hbm(x0p), hbm(x1p), hbm(s0), hbm(d0), hbm(s1), hbm(d1),
        hbm(b0), hbm(b1), hbm(z128), hbm(ones128))

    prob = _tc_dense(px0, py0, pw0, pm0, nc0, px1, py1, pw1, pm1, nc1,
                     Wl1, bl1, Wr1, Wl2, bl2, Wr2, Wc1, bc1, Wc2, bc2)
    return jnp.squeeze(prob, axis=-1)


# speedup vs baseline: 6.5396x; 1.0390x over previous
import functools

import jax
import jax.numpy as jnp
from jax import lax
from jax.experimental import pallas as pl
from jax.experimental.pallas import tpu as pltpu
from jax.experimental.pallas import tpu_sc as plsc

N = 10000
E = 320000
D = 128
G = 16
GP = 32

NSUB = 16
NPT = 640
NP = NSUB * NPT
CH = 128
ECH = 158
EPT = ECH * CH
EP = NSUB * EPT
NCH = NPT // CH
NCH64 = NPT // 64
LR = NP // 128
LRT = NPT // 128


def _sc_passes(x0, x1, s0, d0, s1, d1, b0, b1, z128, ones128):
    mesh = plsc.VectorSubcoreMesh(core_axis_name="c", subcore_axis_name="s")
    f32 = jnp.float32
    outs = (
        jax.ShapeDtypeStruct((NP, D), f32),
        jax.ShapeDtypeStruct((NP, D), f32),
        jax.ShapeDtypeStruct((GP, D), f32),
        jax.ShapeDtypeStruct((GP, D), f32),
        jax.ShapeDtypeStruct((GP, D), f32),
        jax.ShapeDtypeStruct((GP, D), f32),
        jax.ShapeDtypeStruct((GP, D), f32),
        jax.ShapeDtypeStruct((GP, D), f32),
        jax.ShapeDtypeStruct((GP, D), f32),
        jax.ShapeDtypeStruct((GP, D), f32),
        jax.ShapeDtypeStruct((GP, D), f32),
        jax.ShapeDtypeStruct((GP, D), f32),
    )

    @functools.partial(
        pl.kernel,
        mesh=mesh,
        compiler_params=pltpu.CompilerParams(needs_layout_passes=False),
        out_type=outs,
        scratch_types=[
            pltpu.VMEM((2, CH), jnp.int32),
            pltpu.VMEM((2, CH), jnp.int32),
            pltpu.VMEM((CH, D), f32),
            pltpu.VMEM((CH, D), f32),
            pltpu.VMEM((LR, D), f32),
            pltpu.VMEM((LRT, D), f32),
            pltpu.VMEM((NCH, CH), jnp.int32),
            pltpu.VMEM((1, 64), jnp.int32),
            pltpu.VMEM((1, 16), jnp.int32),
            pltpu.VMEM_SHARED((NP, D), f32),
            pltpu.VMEM_SHARED((LR, D), f32),
            pltpu.VMEM_SHARED((GP, D), f32),
            pltpu.VMEM_SHARED((GP, D), f32),
            pltpu.VMEM_SHARED((GP, D), f32),
            pltpu.VMEM_SHARED((GP, D), f32),
            pltpu.VMEM_SHARED((GP, D), f32),
            pltpu.SemaphoreType.DMA,
            pltpu.SemaphoreType.DMA,
        ],
    )
    def body(x0r, x1r, s0r, d0r, s1r, d1r, b0r, b1r, z128r, ones128r,
             y0r, y1r, px0r, py0r, pw0r, pm0r, nc0r,
             px1r, py1r, pw1r, pm1r, nc1r,
             srcv, dstv, rows, rowsB, loc, cntf, bv, idxA, idxB,
             acc, cntW, pxS, pyS, pwS, pmS, ncS, sem, semB):
        c = lax.axis_index("c")
        s = lax.axis_index("s")
        r0 = s * NPT
        i32 = jnp.int32

        def cnt_vec(r):
            ii = jnp.zeros((16,), i32) + r
            return plsc.load_gather(
                cntf, [lax.shift_right_logical(ii, 7), lax.bitwise_and(ii, 127)])

        def divide_rows():
            def rowfix(i, carry):
                dv = jnp.maximum(cnt_vec(carry + i), 1.0)
                for cc in range(D // 16):
                    rows[i, pl.ds(cc * 16, 16)] = rows[i, pl.ds(cc * 16, 16)] / dv
                return carry
            return rowfix

        def zero_acc_slice():
            def zrow(k, carry):
                pltpu.sync_copy(rows, acc.at[pl.ds(r0 + k * CH, CH)])
                return carry
            lax.fori_loop(0, NCH, zrow, 0)

        pltpu.sync_copy(z128r, rows)
        zero_acc_slice()
        pltpu.sync_copy(z128r.at[pl.ds(0, LR)], loc)
        pltpu.sync_copy(z128r.at[pl.ds(0, LRT)], cntf)
        for u in range(4):
            idxA[0, pl.ds(u * 16, 16)] = lax.iota(i32, 16) + (u * 16)
        idxB[0, :] = lax.iota(i32, 16) + 64

        @pl.when(s == 0)
        def _():
            pltpu.sync_copy(rows.at[pl.ds(0, GP)], pxS)
            pltpu.sync_copy(rows.at[pl.ds(0, GP)], pyS)
            pltpu.sync_copy(rows.at[pl.ds(0, GP)], pwS)
            pltpu.sync_copy(rows.at[pl.ds(0, GP)], pmS)
            pltpu.sync_copy(rows.at[pl.ds(0, GP)], ncS)
            pltpu.sync_copy(rows.at[pl.ds(0, LR)], cntW)

        @pl.when(c == 0)
        def _():
            pltpu.sync_copy(b0r.at[s], bv)

        @pl.when(c == 1)
        def _():
            pltpu.sync_copy(b1r.at[s], bv)

        plsc.subcore_barrier()

        ones16 = jnp.ones((16,), f32)

        def hist_slot(slot):
            for u in range(CH // 16):
                d = dstv[slot, pl.ds(u * 16, 16)]
                plsc.addupdate_scatter(
                    loc, [lax.shift_right_logical(d, 7),
                          lax.bitwise_and(d, 127)], ones16)

        def gather_scatter_loop(tab, sr, dr, do_hist):
            pltpu.sync_copy(sr.at[s, 0], srcv.at[0])
            pltpu.sync_copy(dr.at[s, 0], dstv.at[0])
            pltpu.async_copy(tab.at[srcv.at[0]], rows, sem)

            def pair(j2, carry):
                jB = 2 * j2 + 1
                pltpu.sync_copy(sr.at[s, jB], srcv.at[1])
                pltpu.sync_copy(dr.at[s, jB], dstv.at[1])
                pltpu.async_copy(tab.at[srcv.at[1]], rowsB, semB)
                pltpu.make_async_copy(tab.at[srcv.at[0]], rows, sem).wait()
                pltpu.sync_copy(rows, acc.at[dstv.at[0]], add=True)
                if do_hist:
                    hist_slot(0)

                @pl.when(j2 + 1 < ECH // 2)
                def _():
                    pltpu.sync_copy(sr.at[s, jB + 1], srcv.at[0])
                    pltpu.sync_copy(dr.at[s, jB + 1], dstv.at[0])
                    pltpu.async_copy(tab.at[srcv.at[0]], rows, sem)

                pltpu.make_async_copy(tab.at[srcv.at[1]], rowsB, semB).wait()
                pltpu.sync_copy(rowsB, acc.at[dstv.at[1]], add=True)
                if do_hist:
                    hist_slot(1)
                return carry
            lax.fori_loop(0, ECH // 2, pair, 0)

        @pl.when(c == 0)
        def _():
            gather_scatter_loop(x0r, s0r, d0r, True)

        @pl.when(c == 1)
        def _():
            gather_scatter_loop(x1r, s1r, d1r, True)

        pltpu.sync_copy(loc.at[pl.ds(0, 64)], cntW.at[idxA.at[0]], add=True)
        pltpu.sync_copy(loc.at[pl.ds(64, 16)], cntW.at[idxB.at[0]], add=True)
        plsc.subcore_barrier()
        pltpu.sync_copy(cntW.at[pl.ds(LRT * s, LRT)], cntf)

        def p2_loop(xr, yr):
            def p2(k, carry):
                row0 = r0 + k * CH
                pltpu.sync_copy(acc.at[pl.ds(row0, CH)], rows)
                lax.fori_loop(0, CH, divide_rows(), k * CH)

                def mrow(i, carry2):
                    mval = jnp.where(cnt_vec(carry2 + i) > 0.0, 1.0, 0.0)
                    for cc in range(D // 16):
                        rowsB[i, pl.ds(cc * 16, 16)] = mval
                    return carry2
                lax.fori_loop(0, CH, mrow, k * CH)
                pltpu.sync_copy(rows, yr.at[pl.ds(row0, CH)])
                pltpu.sync_copy(rows, pyS.at[bv.at[k]], add=True)
                pltpu.sync_copy(rowsB, pmS.at[bv.at[k]], add=True)
                pltpu.sync_copy(xr.at[pl.ds(row0, CH)], rows)
                pltpu.sync_copy(rows, pxS.at[bv.at[k]], add=True)
                pltpu.sync_copy(ones128r, rows)
                pltpu.sync_copy(rows, ncS.at[bv.at[k]], add=True)
                return carry
            lax.fori_loop(0, NCH, p2, 0)

        @pl.when(c == 0)
        def _():
            p2_loop(x0r, y0r)

        @pl.when(c == 1)
        def _():
            p2_loop(x1r, y1r)

        pltpu.sync_copy(z128r, rows)
        zero_acc_slice()
        plsc.subcore_barrier()

        @pl.when(c == 0)
        def _():
            gather_scatter_loop(y0r, s0r, d0r, False)

        @pl.when(c == 1)
        def _():
            gather_scatter_loop(y1r, s1r, d1r, False)

        plsc.subcore_barrier()

        def p4(k, carry):
            row0 = r0 + k * CH
            pltpu.sync_copy(acc.at[pl.ds(row0, CH)], rows)
            lax.fori_loop(0, CH, divide_rows(), k * CH)
            pltpu.sync_copy(rows, pwS.at[bv.at[k]], add=True)
            return carry
        lax.fori_loop(0, NCH, p4, 0)
        plsc.subcore_barrier()

        def publish(pxo, pyo, pwo, pmo, nco):
            pltpu.sync_copy(pxS, rows.at[pl.ds(0, GP)])
            pltpu.sync_copy(rows.at[pl.ds(0, GP)], pxo)
            pltpu.sync_copy(pyS, rows.at[pl.ds(0, GP)])
            pltpu.sync_copy(rows.at[pl.ds(0, GP)], pyo)
            pltpu.sync_copy(pwS, rows.at[pl.ds(0, GP)])
            pltpu.sync_copy(rows.at[pl.ds(0, GP)], pwo)
            pltpu.sync_copy(pmS, rows.at[pl.ds(0, GP)])
            pltpu.sync_copy(rows.at[pl.ds(0, GP)], pmo)
            pltpu.sync_copy(ncS, rows.at[pl.ds(0, GP)])
            pltpu.sync_copy(rows.at[pl.ds(0, GP)], nco)

        @pl.when((s == 0) & (c == 0))
        def _():
            publish(px0r, py0r, pw0r, pm0r, nc0r)

        @pl.when((s == 0) & (c == 1))
        def _():
            publish(px1r, py1r, pw1r, pm1r, nc1r)

    return body(x0, x1, s0, d0, s1, d1, b0, b1, z128, ones128)


def _mm(a, b):
    return lax.dot_general(a, b, (((1,), (1,)), ((), ())),
                           precision=lax.Precision.HIGHEST,
                           preferred_element_type=jnp.float32)


def _sigmoid(x):
    return 1.0 / (1.0 + jnp.exp(-x))


def _tc_dense(px0, py0, pw0, pm0, nc0, px1, py1, pw1, pm1, nc1,
              Wl1, bl1, Wr1, Wl2, bl2, Wr2, Wc1, bc1, Wc2, bc2):
    def body(px0r, py0r, pw0r, pm0r, nc0r, px1r, py1r, pw1r, pm1r, nc1r,
             Wl1r, bl1r, Wr1r, Wl2r, bl2r, Wr2r, Wc1r, bc1r, Wc2r, bc2r,
             outr):
        A1, B1 = Wl1r[...], bl1r[...]
        R1 = Wr1r[...]
        A2, B2, R2 = Wl2r[...], bl2r[...], Wr2r[...]

        def graph(pxr, pyr, pwr, pmr, ncr):
            nc = ncr[...][:G, 0:1]
            inv = 1.0 / jnp.maximum(nc, 1.0)
            u = jnp.where(nc > 0.0, 1.0, 0.0)
            px = pxr[...][:G, :] * inv
            py = pyr[...][:G, :] * inv
            pw = pwr[...][:G, :] * inv
            pm = pmr[...][:G, 0:1] * inv
            Pf = _mm(py, A1) + _mm(px, R1) + u * B1
            Pz = _mm(pw, A1) + _mm(py, R1) + pm * B1
            return _mm(Pz, A2) + _mm(Pf, R2) + u * B2

        e0 = graph(px0r, py0r, pw0r, pm0r, nc0r)
        e1 = graph(px1r, py1r, pw1r, pm1r, nc1r)
        comb = jnp.concatenate([e0, e1], axis=1)
        h = _sigmoid(_mm(comb, Wc1r[...]) + bc1r[...])
        o = jnp.sum(h * Wc2r[...], axis=1, keepdims=True) + bc2r[...][0, 0]
        outr[...] = _sigmoid(o)

    return pl.pallas_call(
        body,
        out_shape=jax.ShapeDtypeStruct((G, 1), jnp.float32),
    )(px0, py0, pw0, pm0, nc0, px1, py1, pw1, pm1, nc1,
      Wl1, bl1.reshape(1, -1), Wr1, Wl2, bl2.reshape(1, -1), Wr2,
      Wc1, bc1.reshape(1, -1), Wc2, bc2.reshape(1, 1))


def _prep_graph(x, edge_index, batch):
    src = edge_index[0].astype(jnp.int32)
    dst = edge_index[1].astype(jnp.int32)
    srcp = jnp.concatenate([src, jnp.zeros((EP - E,), jnp.int32)]).reshape(NSUB, ECH, CH)
    dstp = jnp.concatenate([dst, jnp.full((EP - E,), N, jnp.int32)]).reshape(NSUB, ECH, CH)
    xp = jnp.concatenate([x, jnp.zeros((NP - N, D), x.dtype)], axis=0)
    bflat = jnp.concatenate([batch.astype(jnp.int32),
                             jnp.full((NP - N,), G, jnp.int32)])
    bp = bflat.reshape(NSUB, NCH, CH)
    return xp, srcp, dstp, bp


def kernel(x0, edge_index0, batch0, x1, edge_index1, batch1,
           Wl1, bl1, Wr1, Wl2, bl2, Wr2, Wc1, bc1, Wc2, bc2):
    x0p, s0, d0, b0 = _prep_graph(x0, edge_index0, batch0)
    x1p, s1, d1, b1 = _prep_graph(x1, edge_index1, batch1)
    z128 = jnp.zeros((CH, D), jnp.float32)
    ones128 = jnp.ones((CH, D), jnp.float32)

    hbm = lambda a: pltpu.with_memory_space_constraint(a, pltpu.HBM)
    (_, _, px0, py0, pw0, pm0, nc0, px1, py1, pw1, pm1, nc1) = _sc_passes(
        hbm(x0p), hbm(x1p), hbm(s0), hbm(d0), hbm(s1), hbm(d1),
        hbm(b0), hbm(b1), hbm(z128), hbm(ones128))

    prob = _tc_dense(px0, py0, pw0, pm0, nc0, px1, py1, pw1, pm1, nc1,
                     Wl1, bl1, Wr1, Wl2, bl2, Wr2, Wc1, bc1, Wc2, bc2)
    return jnp.squeeze(prob, axis=-1)


# speedup vs baseline: 6.5423x; 1.0004x over previous
import functools

import jax
import jax.numpy as jnp
from jax import lax
from jax.experimental import pallas as pl
from jax.experimental.pallas import tpu as pltpu
from jax.experimental.pallas import tpu_sc as plsc

N = 10000
E = 320000
D = 128
G = 16
GP = 32

NSUB = 16
NPT = 640
NP = NSUB * NPT
CH = 128
ECH = 158
EPT = ECH * CH
EP = NSUB * EPT
NCH = NPT // CH
LR = NP // 128
LRT = NPT // 128


def _sc_passes(x0, x1, s0, d0, s1, d1, b0, b1, z128, ones128):
    mesh = plsc.VectorSubcoreMesh(core_axis_name="c", subcore_axis_name="s")
    f32 = jnp.float32
    outs = (
        jax.ShapeDtypeStruct((NP, D), f32),
        jax.ShapeDtypeStruct((NP, D), f32),
        jax.ShapeDtypeStruct((GP, D), f32),
        jax.ShapeDtypeStruct((GP, D), f32),
        jax.ShapeDtypeStruct((GP, D), f32),
        jax.ShapeDtypeStruct((GP, D), f32),
        jax.ShapeDtypeStruct((GP, D), f32),
        jax.ShapeDtypeStruct((GP, D), f32),
        jax.ShapeDtypeStruct((GP, D), f32),
        jax.ShapeDtypeStruct((GP, D), f32),
        jax.ShapeDtypeStruct((GP, D), f32),
        jax.ShapeDtypeStruct((GP, D), f32),
    )

    @functools.partial(
        pl.kernel,
        mesh=mesh,
        compiler_params=pltpu.CompilerParams(needs_layout_passes=False),
        out_type=outs,
        scratch_types=[
            pltpu.VMEM((2, CH), jnp.int32),
            pltpu.VMEM((2, CH), jnp.int32),
            pltpu.VMEM((CH, D), f32),
            pltpu.VMEM((CH, D), f32),
            pltpu.VMEM((LR, D), f32),
            pltpu.VMEM((LRT, D), f32),
            pltpu.VMEM((NCH, CH), jnp.int32),
            pltpu.VMEM((1, 64), jnp.int32),
            pltpu.VMEM((1, 16), jnp.int32),
            pltpu.VMEM_SHARED((NP, D), f32),
            pltpu.VMEM_SHARED((LR, D), f32),
            pltpu.VMEM_SHARED((GP, D), f32),
            pltpu.VMEM_SHARED((GP, D), f32),
            pltpu.VMEM_SHARED((GP, D), f32),
            pltpu.VMEM_SHARED((GP, D), f32),
            pltpu.VMEM_SHARED((GP, D), f32),
            pltpu.SemaphoreType.DMA,
            pltpu.SemaphoreType.DMA,
        ],
    )
    def body(x0r, x1r, s0r, d0r, s1r, d1r, b0r, b1r, z128r, ones128r,
             y0r, y1r, px0r, py0r, pw0r, pm0r, nc0r,
             px1r, py1r, pw1r, pm1r, nc1r,
             srcv, dstv, rows, rowsB, loc, cntf, bv, idxA, idxB,
             acc, cntW, pxS, pyS, pwS, pmS, ncS, sem, semB):
        c = lax.axis_index("c")
        s = lax.axis_index("s")
        r0 = s * NPT
        i32 = jnp.int32

        def cnt_vec(r):
            ii = jnp.zeros((16,), i32) + r
            return plsc.load_gather(
                cntf, [lax.shift_right_logical(ii, 7), lax.bitwise_and(ii, 127)])

        def divide_rows():
            def rowfix(i, carry):
                dv = jnp.maximum(cnt_vec(carry + i), 1.0)
                for cc in range(D // 16):
                    rows[i, pl.ds(cc * 16, 16)] = rows[i, pl.ds(cc * 16, 16)] / dv
                return carry
            return rowfix

        def zero_acc_slice():
            def zrow(k, carry):
                pltpu.sync_copy(rows, acc.at[pl.ds(r0 + k * CH, CH)])
                return carry
            lax.fori_loop(0, NCH, zrow, 0)

        pltpu.sync_copy(z128r, rows)
        zero_acc_slice()
        pltpu.sync_copy(z128r.at[pl.ds(0, LR)], loc)
        pltpu.sync_copy(z128r.at[pl.ds(0, LRT)], cntf)
        for u in range(4):
            idxA[0, pl.ds(u * 16, 16)] = lax.iota(i32, 16) + (u * 16)
        idxB[0, :] = lax.iota(i32, 16) + 64

        @pl.when(s == 0)
        def _():
            pltpu.sync_copy(rows.at[pl.ds(0, GP)], pxS)
            pltpu.sync_copy(rows.at[pl.ds(0, GP)], pyS)
            pltpu.sync_copy(rows.at[pl.ds(0, GP)], pwS)
            pltpu.sync_copy(rows.at[pl.ds(0, GP)], pmS)
            pltpu.sync_copy(rows.at[pl.ds(0, GP)], ncS)
            pltpu.sync_copy(rows.at[pl.ds(0, LR)], cntW)

        @pl.when(c == 0)
        def _():
            pltpu.sync_copy(b0r.at[s], bv)

        @pl.when(c == 1)
        def _():
            pltpu.sync_copy(b1r.at[s], bv)

        plsc.subcore_barrier()

        ones16 = jnp.ones((16,), f32)

        def hist_slot(slot):
            for u in range(CH // 16):
                d = dstv[slot, pl.ds(u * 16, 16)]
                plsc.addupdate_scatter(
                    loc, [lax.shift_right_logical(d, 7),
                          lax.bitwise_and(d, 127)], ones16)

        def gather_scatter_loop(tab, sr, dr, do_hist):
            pltpu.sync_copy(sr.at[s, 0], srcv.at[0])
            pltpu.sync_copy(dr.at[s, 0], dstv.at[0])
            pltpu.async_copy(tab.at[srcv.at[0]], rows, sem)

            def pair(j2, carry):
                jB = 2 * j2 + 1
                pltpu.sync_copy(sr.at[s, jB], srcv.at[1])
                pltpu.sync_copy(dr.at[s, jB], dstv.at[1])
                pltpu.async_copy(tab.at[srcv.at[1]], rowsB, semB)
                pltpu.make_async_copy(tab.at[srcv.at[0]], rows, sem).wait()
                pltpu.sync_copy(rows, acc.at[dstv.at[0]], add=True)
                if do_hist:
                    hist_slot(0)

                @pl.when(j2 + 1 < ECH // 2)
                def _():
                    pltpu.sync_copy(sr.at[s, jB + 1], srcv.at[0])
                    pltpu.sync_copy(dr.at[s, jB + 1], dstv.at[0])
                    pltpu.async_copy(tab.at[srcv.at[0]], rows, sem)

                pltpu.make_async_copy(tab.at[srcv.at[1]], rowsB, semB).wait()
                pltpu.sync_copy(rowsB, acc.at[dstv.at[1]], add=True)
                if do_hist:
                    hist_slot(1)
                return carry
            lax.fori_loop(0, ECH // 2, pair, 0)

        @pl.when(c == 0)
        def _():
            gather_scatter_loop(x0r, s0r, d0r, True)

        @pl.when(c == 1)
        def _():
            gather_scatter_loop(x1r, s1r, d1r, True)

        pltpu.sync_copy(loc.at[pl.ds(0, 64)], cntW.at[idxA.at[0]], add=True)
        pltpu.sync_copy(loc.at[pl.ds(64, 16)], cntW.at[idxB.at[0]], add=True)
        plsc.subcore_barrier()
        pltpu.sync_copy(cntW.at[pl.ds(LRT * s, LRT)], cntf)

        def p2_loop(xr, yr):
            def p2(k, carry):
                row0 = r0 + k * CH
                pltpu.sync_copy(acc.at[pl.ds(row0, CH)], rows)
                lax.fori_loop(0, CH, divide_rows(), k * CH)

                def mrow(i, carry2):
                    mval = jnp.where(cnt_vec(carry2 + i) > 0.0, 1.0, 0.0)
                    for cc in range(D // 16):
                        rowsB[i, pl.ds(cc * 16, 16)] = mval
                    return carry2
                lax.fori_loop(0, CH, mrow, k * CH)
                pltpu.sync_copy(rows, yr.at[pl.ds(row0, CH)])
                pltpu.sync_copy(rows, pyS.at[bv.at[k]], add=True)
                pltpu.sync_copy(rowsB, pmS.at[bv.at[k]], add=True)
                pltpu.sync_copy(xr.at[pl.ds(row0, CH)], rows)
                pltpu.sync_copy(rows, pxS.at[bv.at[k]], add=True)
                pltpu.sync_copy(ones128r, rows)
                pltpu.sync_copy(rows, ncS.at[bv.at[k]], add=True)
                return carry
            lax.fori_loop(0, NCH, p2, 0)

        @pl.when(c == 0)
        def _():
            p2_loop(x0r, y0r)

        @pl.when(c == 1)
        def _():
            p2_loop(x1r, y1r)

        pltpu.sync_copy(z128r, rows)
        zero_acc_slice()
        plsc.subcore_barrier()

        @pl.when(c == 0)
        def _():
            gather_scatter_loop(y0r, s0r, d0r, False)

        @pl.when(c == 1)
        def _():
            gather_scatter_loop(y1r, s1r, d1r, False)

        plsc.subcore_barrier()

        def p4(k, carry):
            row0 = r0 + k * CH
            pltpu.sync_copy(acc.at[pl.ds(row0, CH)], rows)
            lax.fori_loop(0, CH, divide_rows(), k * CH)
            pltpu.sync_copy(rows, pwS.at[bv.at[k]], add=True)
            return carry
        lax.fori_loop(0, NCH, p4, 0)
        plsc.subcore_barrier()

        def publish(pxo, pyo, pwo, pmo, nco):
            pltpu.sync_copy(pxS, rows.at[pl.ds(0, GP)])
            pltpu.sync_copy(rows.at[pl.ds(0, GP)], pxo)
            pltpu.sync_copy(pyS, rows.at[pl.ds(0, GP)])
            pltpu.sync_copy(rows.at[pl.ds(0, GP)], pyo)
            pltpu.sync_copy(pwS, rows.at[pl.ds(0, GP)])
            pltpu.sync_copy(rows.at[pl.ds(0, GP)], pwo)
            pltpu.sync_copy(pmS, rows.at[pl.ds(0, GP)])
            pltpu.sync_copy(rows.at[pl.ds(0, GP)], pmo)
            pltpu.sync_copy(ncS, rows.at[pl.ds(0, GP)])
            pltpu.sync_copy(rows.at[pl.ds(0, GP)], nco)

        @pl.when((s == 0) & (c == 0))
        def _():
            publish(px0r, py0r, pw0r, pm0r, nc0r)

        @pl.when((s == 0) & (c == 1))
        def _():
            publish(px1r, py1r, pw1r, pm1r, nc1r)

    return body(x0, x1, s0, d0, s1, d1, b0, b1, z128, ones128)


def _mm(a, b):
    return lax.dot_general(a, b, (((1,), (1,)), ((), ())),
                           precision=lax.Precision.HIGHEST,
                           preferred_element_type=jnp.float32)


def _sigmoid(x):
    return 1.0 / (1.0 + jnp.exp(-x))


def _tc_dense(px0, py0, pw0, pm0, nc0, px1, py1, pw1, pm1, nc1,
              Wl1, bl1, Wr1, Wl2, bl2, Wr2, Wc1, bc1, Wc2, bc2):
    def body(px0r, py0r, pw0r, pm0r, nc0r, px1r, py1r, pw1r, pm1r, nc1r,
             Wl1r, bl1r, Wr1r, Wl2r, bl2r, Wr2r, Wc1r, bc1r, Wc2r, bc2r,
             outr):
        A1, B1 = Wl1r[...], bl1r[...]
        R1 = Wr1r[...]
        A2, B2, R2 = Wl2r[...], bl2r[...], Wr2r[...]

        def graph(pxr, pyr, pwr, pmr, ncr):
            nc = ncr[...][:G, 0:1]
            inv = 1.0 / jnp.maximum(nc, 1.0)
            u = jnp.where(nc > 0.0, 1.0, 0.0)
            px = pxr[...][:G, :] * inv
            py = pyr[...][:G, :] * inv
            pw = pwr[...][:G, :] * inv
            pm = pmr[...][:G, 0:1] * inv
            Pf = _mm(py, A1) + _mm(px, R1) + u * B1
            Pz = _mm(pw, A1) + _mm(py, R1) + pm * B1
            return _mm(Pz, A2) + _mm(Pf, R2) + u * B2

        e0 = graph(px0r, py0r, pw0r, pm0r, nc0r)
        e1 = graph(px1r, py1r, pw1r, pm1r, nc1r)
        comb = jnp.concatenate([e0, e1], axis=1)
        h = _sigmoid(_mm(comb, Wc1r[...]) + bc1r[...])
        o = jnp.sum(h * Wc2r[...], axis=1, keepdims=True) + bc2r[...][0, 0]
        outr[...] = _sigmoid(o)

    return pl.pallas_call(
        body,
        out_shape=jax.ShapeDtypeStruct((G, 1), jnp.float32),
    )(px0, py0, pw0, pm0, nc0, px1, py1, pw1, pm1, nc1,
      Wl1, bl1.reshape(1, -1), Wr1, Wl2, bl2.reshape(1, -1), Wr2,
      Wc1, bc1.reshape(1, -1), Wc2, bc2.reshape(1, 1))


def _prep_graph(x, edge_index, batch):
    src = edge_index[0].astype(jnp.int32)
    dst = edge_index[1].astype(jnp.int32)
    srcp = jnp.concatenate([src, jnp.zeros((EP - E,), jnp.int32)]).reshape(NSUB, ECH, CH)
    dstp = jnp.concatenate([dst, jnp.full((EP - E,), N, jnp.int32)]).reshape(NSUB, ECH, CH)
    xp = jnp.concatenate([x, jnp.zeros((NP - N, D), x.dtype)], axis=0)
    bflat = jnp.concatenate([batch.astype(jnp.int32),
                             jnp.full((NP - N,), G, jnp.int32)])
    bp = bflat.reshape(NSUB, NCH, CH)
    return xp, srcp, dstp, bp


def kernel(x0, edge_index0, batch0, x1, edge_index1, batch1,
           Wl1, bl1, Wr1, Wl2, bl2, Wr2, Wc1, bc1, Wc2, bc2):
    x0p, s0, d0, b0 = _prep_graph(x0, edge_index0, batch0)
    x1p, s1, d1, b1 = _prep_graph(x1, edge_index1, batch1)
    z128 = jnp.zeros((CH, D), jnp.float32)
    ones128 = jnp.ones((CH, D), jnp.float32)

    hbm = lambda a: pltpu.with_memory_space_constraint(a, pltpu.HBM)
    (_, _, px0, py0, pw0, pm0, nc0, px1, py1, pw1, pm1, nc1) = _sc_passes(
        hbm(x0p), hbm(x1p), hbm(s0), hbm(d0), hbm(s1), hbm(d1),
        hbm(b0), hbm(b1), hbm(z128), hbm(ones128))

    prob = _tc_dense(px0, py0, pw0, pm0, nc0, px1, py1, pw1, pm1, nc1,
                     Wl1, bl1, Wr1, Wl2, bl2, Wr2, Wc1, bc1, Wc2, bc2)
    return jnp.squeeze(prob, axis=-1)


# speedup vs baseline: 7.5127x; 1.1483x over previous
import functools

import jax
import jax.numpy as jnp
from jax import lax
from jax.experimental import pallas as pl
from jax.experimental.pallas import tpu as pltpu
from jax.experimental.pallas import tpu_sc as plsc

N = 10000
E = 320000
D = 128
G = 16
GP = 32

NSUB = 16
NPT = 640
NP = NSUB * NPT
CH = 128
ECH = 158
EPT = ECH * CH
EP = NSUB * EPT
NCH = NPT // CH
LR = NP // 128
LRT = NPT // 128


def _sc_passes(x0, x1, s0, d0, s1, d1, b0, b1, z128, ones128):
    mesh = plsc.VectorSubcoreMesh(core_axis_name="c", subcore_axis_name="s")
    f32 = jnp.float32
    outs = (
        jax.ShapeDtypeStruct((NP, D), f32),
        jax.ShapeDtypeStruct((NP, D), f32),
        jax.ShapeDtypeStruct((GP, D), f32),
        jax.ShapeDtypeStruct((GP, D), f32),
        jax.ShapeDtypeStruct((GP, D), f32),
        jax.ShapeDtypeStruct((GP, D), f32),
        jax.ShapeDtypeStruct((GP, D), f32),
        jax.ShapeDtypeStruct((GP, D), f32),
        jax.ShapeDtypeStruct((GP, D), f32),
        jax.ShapeDtypeStruct((GP, D), f32),
        jax.ShapeDtypeStruct((GP, D), f32),
        jax.ShapeDtypeStruct((GP, D), f32),
    )

    @functools.partial(
        pl.kernel,
        mesh=mesh,
        compiler_params=pltpu.CompilerParams(needs_layout_passes=False),
        out_type=outs,
        scratch_types=[
            pltpu.VMEM((2, 2, CH), jnp.int32),
            pltpu.VMEM((2, 2, CH), jnp.int32),
            pltpu.VMEM((CH, D), f32),
            pltpu.VMEM((CH, D), f32),
            pltpu.VMEM((LR, D), f32),
            pltpu.VMEM((LRT, D), f32),
            pltpu.VMEM((NCH, CH), jnp.int32),
            pltpu.VMEM((1, 64), jnp.int32),
            pltpu.VMEM((1, 16), jnp.int32),
            pltpu.VMEM_SHARED((NP, D), f32),
            pltpu.VMEM_SHARED((LR, D), f32),
            pltpu.VMEM_SHARED((GP, D), f32),
            pltpu.VMEM_SHARED((GP, D), f32),
            pltpu.VMEM_SHARED((GP, D), f32),
            pltpu.VMEM_SHARED((GP, D), f32),
            pltpu.VMEM_SHARED((GP, D), f32),
            pltpu.SemaphoreType.DMA,
            pltpu.SemaphoreType.DMA,
            pltpu.SemaphoreType.DMA,
        ],
    )
    def body(x0r, x1r, s0r, d0r, s1r, d1r, b0r, b1r, z128r, ones128r,
             y0r, y1r, px0r, py0r, pw0r, pm0r, nc0r,
             px1r, py1r, pw1r, pm1r, nc1r,
             sidx, didx, rows, rowsB, loc, cntf, bv, idxA, idxB,
             acc, cntW, pxS, pyS, pwS, pmS, ncS, sem, semB, semI):
        c = lax.axis_index("c")
        s = lax.axis_index("s")
        r0 = s * NPT
        i32 = jnp.int32

        def cnt_vec(r):
            ii = jnp.zeros((16,), i32) + r
            return plsc.load_gather(
                cntf, [lax.shift_right_logical(ii, 7), lax.bitwise_and(ii, 127)])

        def divide_rows():
            def rowfix(i, carry):
                dv = jnp.maximum(cnt_vec(carry + i), 1.0)
                for cc in range(D // 16):
                    rows[i, pl.ds(cc * 16, 16)] = rows[i, pl.ds(cc * 16, 16)] / dv
                return carry
            return rowfix

        def zero_acc_slice():
            def zrow(k, carry):
                pltpu.sync_copy(rows, acc.at[pl.ds(r0 + k * CH, CH)])
                return carry
            lax.fori_loop(0, NCH, zrow, 0)

        pltpu.sync_copy(z128r, rows)
        zero_acc_slice()
        pltpu.sync_copy(z128r.at[pl.ds(0, LR)], loc)
        pltpu.sync_copy(z128r.at[pl.ds(0, LRT)], cntf)
        for u in range(4):
            idxA[0, pl.ds(u * 16, 16)] = lax.iota(i32, 16) + (u * 16)
        idxB[0, :] = lax.iota(i32, 16) + 64

        @pl.when(s == 0)
        def _():
            pltpu.sync_copy(rows.at[pl.ds(0, GP)], pxS)
            pltpu.sync_copy(rows.at[pl.ds(0, GP)], pyS)
            pltpu.sync_copy(rows.at[pl.ds(0, GP)], pwS)
            pltpu.sync_copy(rows.at[pl.ds(0, GP)], pmS)
            pltpu.sync_copy(rows.at[pl.ds(0, GP)], ncS)
            pltpu.sync_copy(rows.at[pl.ds(0, LR)], cntW)

        @pl.when(c == 0)
        def _():
            pltpu.sync_copy(b0r.at[s], bv)

        @pl.when(c == 1)
        def _():
            pltpu.sync_copy(b1r.at[s], bv)

        plsc.subcore_barrier()

        ones16 = jnp.ones((16,), f32)

        def hist_slot(q, phase):
            for u in range(CH // 16):
                d = didx[q, phase, pl.ds(u * 16, 16)]
                plsc.addupdate_scatter(
                    loc, [lax.shift_right_logical(d, 7),
                          lax.bitwise_and(d, 127)], ones16)

        NPAIR = ECH // 2

        def gather_scatter_loop(tab, sr, dr, do_hist):
            pltpu.sync_copy(sr.at[s, pl.ds(0, 2)], sidx.at[0])
            pltpu.sync_copy(dr.at[s, pl.ds(0, 2)], didx.at[0])
            pltpu.async_copy(tab.at[sidx.at[0, 0]], rows, sem)

            def pair(j2, carry):
                q = j2 % 2
                qn = 1 - q
                pltpu.async_copy(tab.at[sidx.at[q, 1]], rowsB, semB)

                @pl.when(j2 + 1 < NPAIR)
                def _():
                    pltpu.async_copy(sr.at[s, pl.ds(2 * (j2 + 1), 2)],
                                     sidx.at[qn], semI)
                    pltpu.async_copy(dr.at[s, pl.ds(2 * (j2 + 1), 2)],
                                     didx.at[qn], semI)

                pltpu.make_async_copy(tab.at[sidx.at[q, 0]], rows, sem).wait()
                pltpu.sync_copy(rows, acc.at[didx.at[q, 0]], add=True)
                if do_hist:
                    hist_slot(q, 0)

                @pl.when(j2 + 1 < NPAIR)
                def _():
                    pltpu.make_async_copy(sr.at[s, pl.ds(0, 2)],
                                          sidx.at[qn], semI).wait()
                    pltpu.make_async_copy(dr.at[s, pl.ds(0, 2)],
                                          didx.at[qn], semI).wait()
                    pltpu.async_copy(tab.at[sidx.at[qn, 0]], rows, sem)

                pltpu.make_async_copy(tab.at[sidx.at[q, 1]], rowsB, semB).wait()
                pltpu.sync_copy(rowsB, acc.at[didx.at[q, 1]], add=True)
                if do_hist:
                    hist_slot(q, 1)
                return carry
            lax.fori_loop(0, NPAIR, pair, 0)

        @pl.when(c == 0)
        def _():
            gather_scatter_loop(x0r, s0r, d0r, True)

        @pl.when(c == 1)
        def _():
            gather_scatter_loop(x1r, s1r, d1r, True)

        pltpu.sync_copy(loc.at[pl.ds(0, 64)], cntW.at[idxA.at[0]], add=True)
        pltpu.sync_copy(loc.at[pl.ds(64, 16)], cntW.at[idxB.at[0]], add=True)
        plsc.subcore_barrier()
        pltpu.sync_copy(cntW.at[pl.ds(LRT * s, LRT)], cntf)

        def p2_loop(xr, yr):
            def p2(k, carry):
                row0 = r0 + k * CH
                pltpu.sync_copy(acc.at[pl.ds(row0, CH)], rows)
                lax.fori_loop(0, CH, divide_rows(), k * CH)

                def mrow(i, carry2):
                    mval = jnp.where(cnt_vec(carry2 + i) > 0.0, 1.0, 0.0)
                    for cc in range(D // 16):
                        rowsB[i, pl.ds(cc * 16, 16)] = mval
                    return carry2
                lax.fori_loop(0, CH, mrow, k * CH)
                pltpu.sync_copy(rows, yr.at[pl.ds(row0, CH)])
                pltpu.sync_copy(rows, pyS.at[bv.at[k]], add=True)
                pltpu.sync_copy(rowsB, pmS.at[bv.at[k]], add=True)
                pltpu.sync_copy(xr.at[pl.ds(row0, CH)], rows)
                pltpu.sync_copy(rows, pxS.at[bv.at[k]], add=True)
                pltpu.sync_copy(ones128r, rows)
                pltpu.sync_copy(rows, ncS.at[bv.at[k]], add=True)
                return carry
            lax.fori_loop(0, NCH, p2, 0)

        @pl.when(c == 0)
        def _():
            p2_loop(x0r, y0r)

        @pl.when(c == 1)
        def _():
            p2_loop(x1r, y1r)

        pltpu.sync_copy(z128r, rows)
        zero_acc_slice()
        plsc.subcore_barrier()

        @pl.when(c == 0)
        def _():
            gather_scatter_loop(y0r, s0r, d0r, False)

        @pl.when(c == 1)
        def _():
            gather_scatter_loop(y1r, s1r, d1r, False)

        plsc.subcore_barrier()

        def p4(k, carry):
            row0 = r0 + k * CH
            pltpu.sync_copy(acc.at[pl.ds(row0, CH)], rows)
            lax.fori_loop(0, CH, divide_rows(), k * CH)
            pltpu.sync_copy(rows, pwS.at[bv.at[k]], add=True)
            return carry
        lax.fori_loop(0, NCH, p4, 0)
        plsc.subcore_barrier()

        def publish(pxo, pyo, pwo, pmo, nco):
            pltpu.sync_copy(pxS, rows.at[pl.ds(0, GP)])
            pltpu.sync_copy(rows.at[pl.ds(0, GP)], pxo)
            pltpu.sync_copy(pyS, rows.at[pl.ds(0, GP)])
            pltpu.sync_copy(rows.at[pl.ds(0, GP)], pyo)
            pltpu.sync_copy(pwS, rows.at[pl.ds(0, GP)])
            pltpu.sync_copy(rows.at[pl.ds(0, GP)], pwo)
            pltpu.sync_copy(pmS, rows.at[pl.ds(0, GP)])
            pltpu.sync_copy(rows.at[pl.ds(0, GP)], pmo)
            pltpu.sync_copy(ncS, rows.at[pl.ds(0, GP)])
            pltpu.sync_copy(rows.at[pl.ds(0, GP)], nco)

        @pl.when((s == 0) & (c == 0))
        def _():
            publish(px0r, py0r, pw0r, pm0r, nc0r)

        @pl.when((s == 0) & (c == 1))
        def _():
            publish(px1r, py1r, pw1r, pm1r, nc1r)

    return body(x0, x1, s0, d0, s1, d1, b0, b1, z128, ones128)


def _mm(a, b):
    return lax.dot_general(a, b, (((1,), (1,)), ((), ())),
                           precision=lax.Precision.HIGHEST,
                           preferred_element_type=jnp.float32)


def _sigmoid(x):
    return 1.0 / (1.0 + jnp.exp(-x))


def _tc_dense(px0, py0, pw0, pm0, nc0, px1, py1, pw1, pm1, nc1,
              Wl1, bl1, Wr1, Wl2, bl2, Wr2, Wc1, bc1, Wc2, bc2):
    def body(px0r, py0r, pw0r, pm0r, nc0r, px1r, py1r, pw1r, pm1r, nc1r,
             Wl1r, bl1r, Wr1r, Wl2r, bl2r, Wr2r, Wc1r, bc1r, Wc2r, bc2r,
             outr):
        A1, B1 = Wl1r[...], bl1r[...]
        R1 = Wr1r[...]
        A2, B2, R2 = Wl2r[...], bl2r[...], Wr2r[...]

        def graph(pxr, pyr, pwr, pmr, ncr):
            nc = ncr[...][:G, 0:1]
            inv = 1.0 / jnp.maximum(nc, 1.0)
            u = jnp.where(nc > 0.0, 1.0, 0.0)
            px = pxr[...][:G, :] * inv
            py = pyr[...][:G, :] * inv
            pw = pwr[...][:G, :] * inv
            pm = pmr[...][:G, 0:1] * inv
            Pf = _mm(py, A1) + _mm(px, R1) + u * B1
            Pz = _mm(pw, A1) + _mm(py, R1) + pm * B1
            return _mm(Pz, A2) + _mm(Pf, R2) + u * B2

        e0 = graph(px0r, py0r, pw0r, pm0r, nc0r)
        e1 = graph(px1r, py1r, pw1r, pm1r, nc1r)
        comb = jnp.concatenate([e0, e1], axis=1)
        h = _sigmoid(_mm(comb, Wc1r[...]) + bc1r[...])
        o = jnp.sum(h * Wc2r[...], axis=1, keepdims=True) + bc2r[...][0, 0]
        outr[...] = _sigmoid(o)

    return pl.pallas_call(
        body,
        out_shape=jax.ShapeDtypeStruct((G, 1), jnp.float32),
    )(px0, py0, pw0, pm0, nc0, px1, py1, pw1, pm1, nc1,
      Wl1, bl1.reshape(1, -1), Wr1, Wl2, bl2.reshape(1, -1), Wr2,
      Wc1, bc1.reshape(1, -1), Wc2, bc2.reshape(1, 1))


def _prep_graph(x, edge_index, batch):
    src = edge_index[0].astype(jnp.int32)
    dst = edge_index[1].astype(jnp.int32)
    srcp = jnp.concatenate([src, jnp.zeros((EP - E,), jnp.int32)]).reshape(NSUB, ECH, CH)
    dstp = jnp.concatenate([dst, jnp.full((EP - E,), N, jnp.int32)]).reshape(NSUB, ECH, CH)
    xp = jnp.concatenate([x, jnp.zeros((NP - N, D), x.dtype)], axis=0)
    bflat = jnp.concatenate([batch.astype(jnp.int32),
                             jnp.full((NP - N,), G, jnp.int32)])
    bp = bflat.reshape(NSUB, NCH, CH)
    return xp, srcp, dstp, bp


def kernel(x0, edge_index0, batch0, x1, edge_index1, batch1,
           Wl1, bl1, Wr1, Wl2, bl2, Wr2, Wc1, bc1, Wc2, bc2):
    x0p, s0, d0, b0 = _prep_graph(x0, edge_index0, batch0)
    x1p, s1, d1, b1 = _prep_graph(x1, edge_index1, batch1)
    z128 = jnp.zeros((CH, D), jnp.float32)
    ones128 = jnp.ones((CH, D), jnp.float32)

    hbm = lambda a: pltpu.with_memory_space_constraint(a, pltpu.HBM)
    (_, _, px0, py0, pw0, pm0, nc0, px1, py1, pw1, pm1, nc1) = _sc_passes(
        hbm(x0p), hbm(x1p), hbm(s0), hbm(d0), hbm(s1), hbm(d1),
        hbm(b0), hbm(b1), hbm(z128), hbm(ones128))

    prob = _tc_dense(px0, py0, pw0, pm0, nc0, px1, py1, pw1, pm1, nc1,
                     Wl1, bl1, Wr1, Wl2, bl2, Wr2, Wc1, bc1, Wc2, bc2)
    return jnp.squeeze(prob, axis=-1)


# speedup vs baseline: 7.5603x; 1.0063x over previous
import functools

import jax
import jax.numpy as jnp
from jax import lax
from jax.experimental import pallas as pl
from jax.experimental.pallas import tpu as pltpu
from jax.experimental.pallas import tpu_sc as plsc

N = 10000
E = 320000
D = 128
G = 16
GP = 32

NSUB = 16
NPT = 640
NP = NSUB * NPT
CH = 128
ECH = 158
EPT = ECH * CH
EP = NSUB * EPT
NCH = NPT // CH
LR = NP // 128
LRT = NPT // 128


def _sc_passes(x0, x1, s0, d0, s1, d1, b0, b1, z128, ones128):
    mesh = plsc.VectorSubcoreMesh(core_axis_name="c", subcore_axis_name="s")
    f32 = jnp.float32
    outs = (
        jax.ShapeDtypeStruct((NP, D), f32),
        jax.ShapeDtypeStruct((NP, D), f32),
        jax.ShapeDtypeStruct((GP, D), f32),
        jax.ShapeDtypeStruct((GP, D), f32),
        jax.ShapeDtypeStruct((GP, D), f32),
        jax.ShapeDtypeStruct((GP, D), f32),
        jax.ShapeDtypeStruct((GP, D), f32),
        jax.ShapeDtypeStruct((GP, D), f32),
        jax.ShapeDtypeStruct((GP, D), f32),
        jax.ShapeDtypeStruct((GP, D), f32),
        jax.ShapeDtypeStruct((GP, D), f32),
        jax.ShapeDtypeStruct((GP, D), f32),
    )

    @functools.partial(
        pl.kernel,
        mesh=mesh,
        compiler_params=pltpu.CompilerParams(needs_layout_passes=False),
        out_type=outs,
        scratch_types=[
            pltpu.VMEM((2, 2, CH), jnp.int32),
            pltpu.VMEM((2, 2, CH), jnp.int32),
            pltpu.VMEM((CH, D), f32),
            pltpu.VMEM((CH, D), f32),
            pltpu.VMEM((LR, D), f32),
            pltpu.VMEM((LRT, D), f32),
            pltpu.VMEM((NCH, CH), jnp.int32),
            pltpu.VMEM((1, 64), jnp.int32),
            pltpu.VMEM((1, 16), jnp.int32),
            pltpu.VMEM_SHARED((NP, D), f32),
            pltpu.VMEM_SHARED((LR, D), f32),
            pltpu.VMEM_SHARED((GP, D), f32),
            pltpu.VMEM_SHARED((GP, D), f32),
            pltpu.VMEM_SHARED((GP, D), f32),
            pltpu.VMEM_SHARED((GP, D), f32),
            pltpu.VMEM_SHARED((GP, D), f32),
            pltpu.SemaphoreType.DMA,
            pltpu.SemaphoreType.DMA,
            pltpu.SemaphoreType.DMA,
            pltpu.SemaphoreType.DMA,
            pltpu.SemaphoreType.DMA,
        ],
    )
    def body(x0r, x1r, s0r, d0r, s1r, d1r, b0r, b1r, z128r, ones128r,
             y0r, y1r, px0r, py0r, pw0r, pm0r, nc0r,
             px1r, py1r, pw1r, pm1r, nc1r,
             sidx, didx, rows, rowsB, loc, cntf, bv, idxA, idxB,
             acc, cntW, pxS, pyS, pwS, pmS, ncS, sem, semB, semI, semS, semT):
        c = lax.axis_index("c")
        s = lax.axis_index("s")
        r0 = s * NPT
        i32 = jnp.int32

        def cnt_vec(r):
            ii = jnp.zeros((16,), i32) + r
            return plsc.load_gather(
                cntf, [lax.shift_right_logical(ii, 7), lax.bitwise_and(ii, 127)])

        def divide_rows():
            def rowfix(i, carry):
                dv = jnp.maximum(cnt_vec(carry + i), 1.0)
                for cc in range(D // 16):
                    rows[i, pl.ds(cc * 16, 16)] = rows[i, pl.ds(cc * 16, 16)] / dv
                return carry
            return rowfix

        def zero_acc_slice():
            def zrow(k, carry):
                pltpu.sync_copy(rows, acc.at[pl.ds(r0 + k * CH, CH)])
                return carry
            lax.fori_loop(0, NCH, zrow, 0)

        pltpu.sync_copy(z128r, rows)
        zero_acc_slice()
        pltpu.sync_copy(z128r.at[pl.ds(0, LR)], loc)
        pltpu.sync_copy(z128r.at[pl.ds(0, LRT)], cntf)
        for u in range(4):
            idxA[0, pl.ds(u * 16, 16)] = lax.iota(i32, 16) + (u * 16)
        idxB[0, :] = lax.iota(i32, 16) + 64

        @pl.when(s == 0)
        def _():
            pltpu.sync_copy(rows.at[pl.ds(0, GP)], pxS)
            pltpu.sync_copy(rows.at[pl.ds(0, GP)], pyS)
            pltpu.sync_copy(rows.at[pl.ds(0, GP)], pwS)
            pltpu.sync_copy(rows.at[pl.ds(0, GP)], pmS)
            pltpu.sync_copy(rows.at[pl.ds(0, GP)], ncS)
            pltpu.sync_copy(rows.at[pl.ds(0, LR)], cntW)

        @pl.when(c == 0)
        def _():
            pltpu.sync_copy(b0r.at[s], bv)

        @pl.when(c == 1)
        def _():
            pltpu.sync_copy(b1r.at[s], bv)

        plsc.subcore_barrier()

        ones16 = jnp.ones((16,), f32)

        def hist_slot(q, phase):
            for u in range(CH // 16):
                d = didx[q, phase, pl.ds(u * 16, 16)]
                plsc.addupdate_scatter(
                    loc, [lax.shift_right_logical(d, 7),
                          lax.bitwise_and(d, 127)], ones16)

        NPAIR = ECH // 2

        def gather_scatter_loop(tab, sr, dr, do_hist):
            pltpu.sync_copy(sr.at[s, pl.ds(0, 2)], sidx.at[0])
            pltpu.sync_copy(dr.at[s, pl.ds(0, 2)], didx.at[0])
            pltpu.async_copy(tab.at[sidx.at[0, 0]], rows, sem)

            def pair(j2, carry):
                q = j2 % 2
                qn = 1 - q

                @pl.when(j2 > 0)
                def _():
                    pltpu.make_async_copy(rowsB, acc.at[didx.at[q, 1]],
                                          semT).wait()

                pltpu.async_copy(tab.at[sidx.at[q, 1]], rowsB, semB)

                @pl.when(j2 + 1 < NPAIR)
                def _():
                    pltpu.async_copy(sr.at[s, pl.ds(2 * (j2 + 1), 2)],
                                     sidx.at[qn], semI)
                    pltpu.async_copy(dr.at[s, pl.ds(2 * (j2 + 1), 2)],
                                     didx.at[qn], semI)

                pltpu.make_async_copy(tab.at[sidx.at[q, 0]], rows, sem).wait()
                pltpu.async_copy(rows, acc.at[didx.at[q, 0]], semS, add=True)
                if do_hist:
                    hist_slot(q, 0)

                @pl.when(j2 + 1 < NPAIR)
                def _():
                    pltpu.make_async_copy(sr.at[s, pl.ds(0, 2)],
                                          sidx.at[qn], semI).wait()
                    pltpu.make_async_copy(dr.at[s, pl.ds(0, 2)],
                                          didx.at[qn], semI).wait()
                    pltpu.make_async_copy(rows, acc.at[didx.at[q, 0]],
                                          semS).wait()
                    pltpu.async_copy(tab.at[sidx.at[qn, 0]], rows, sem)

                pltpu.make_async_copy(tab.at[sidx.at[q, 1]], rowsB, semB).wait()
                pltpu.async_copy(rowsB, acc.at[didx.at[q, 1]], semT, add=True)
                if do_hist:
                    hist_slot(q, 1)
                return carry
            lax.fori_loop(0, NPAIR, pair, 0)
            pltpu.make_async_copy(rows, acc.at[didx.at[0, 0]], semS).wait()
            pltpu.make_async_copy(rowsB, acc.at[didx.at[0, 1]], semT).wait()

        @pl.when(c == 0)
        def _():
            gather_scatter_loop(x0r, s0r, d0r, True)

        @pl.when(c == 1)
        def _():
            gather_scatter_loop(x1r, s1r, d1r, True)

        pltpu.sync_copy(loc.at[pl.ds(0, 64)], cntW.at[idxA.at[0]], add=True)
        pltpu.sync_copy(loc.at[pl.ds(64, 16)], cntW.at[idxB.at[0]], add=True)
        plsc.subcore_barrier()
        pltpu.sync_copy(cntW.at[pl.ds(LRT * s, LRT)], cntf)

        def p2_loop(xr, yr):
            def p2(k, carry):
                row0 = r0 + k * CH
                pltpu.sync_copy(acc.at[pl.ds(row0, CH)], rows)
                lax.fori_loop(0, CH, divide_rows(), k * CH)

                def mrow(i, carry2):
                    mval = jnp.where(cnt_vec(carry2 + i) > 0.0, 1.0, 0.0)
                    for cc in range(D // 16):
                        rowsB[i, pl.ds(cc * 16, 16)] = mval
                    return carry2
                lax.fori_loop(0, CH, mrow, k * CH)
                pltpu.sync_copy(rows, yr.at[pl.ds(row0, CH)])
                pltpu.sync_copy(rows, pyS.at[bv.at[k]], add=True)
                pltpu.sync_copy(rowsB, pmS.at[bv.at[k]], add=True)
                pltpu.sync_copy(xr.at[pl.ds(row0, CH)], rows)
                pltpu.sync_copy(rows, pxS.at[bv.at[k]], add=True)
                pltpu.sync_copy(ones128r, rows)
                pltpu.sync_copy(rows, ncS.at[bv.at[k]], add=True)
                return carry
            lax.fori_loop(0, NCH, p2, 0)

        @pl.when(c == 0)
        def _():
            p2_loop(x0r, y0r)

        @pl.when(c == 1)
        def _():
            p2_loop(x1r, y1r)

        pltpu.sync_copy(z128r, rows)
        zero_acc_slice()
        plsc.subcore_barrier()

        @pl.when(c == 0)
        def _():
            gather_scatter_loop(y0r, s0r, d0r, False)

        @pl.when(c == 1)
        def _():
            gather_scatter_loop(y1r, s1r, d1r, False)

        plsc.subcore_barrier()

        def p4(k, carry):
            row0 = r0 + k * CH
            pltpu.sync_copy(acc.at[pl.ds(row0, CH)], rows)
            lax.fori_loop(0, CH, divide_rows(), k * CH)
            pltpu.sync_copy(rows, pwS.at[bv.at[k]], add=True)
            return carry
        lax.fori_loop(0, NCH, p4, 0)
        plsc.subcore_barrier()

        def publish(pxo, pyo, pwo, pmo, nco):
            pltpu.sync_copy(pxS, rows.at[pl.ds(0, GP)])
            pltpu.sync_copy(rows.at[pl.ds(0, GP)], pxo)
            pltpu.sync_copy(pyS, rows.at[pl.ds(0, GP)])
            pltpu.sync_copy(rows.at[pl.ds(0, GP)], pyo)
            pltpu.sync_copy(pwS, rows.at[pl.ds(0, GP)])
            pltpu.sync_copy(rows.at[pl.ds(0, GP)], pwo)
            pltpu.sync_copy(pmS, rows.at[pl.ds(0, GP)])
            pltpu.sync_copy(rows.at[pl.ds(0, GP)], pmo)
            pltpu.sync_copy(ncS, rows.at[pl.ds(0, GP)])
            pltpu.sync_copy(rows.at[pl.ds(0, GP)], nco)

        @pl.when((s == 0) & (c == 0))
        def _():
            publish(px0r, py0r, pw0r, pm0r, nc0r)

        @pl.when((s == 0) & (c == 1))
        def _():
            publish(px1r, py1r, pw1r, pm1r, nc1r)

    return body(x0, x1, s0, d0, s1, d1, b0, b1, z128, ones128)


def _mm(a, b):
    return lax.dot_general(a, b, (((1,), (1,)), ((), ())),
                           precision=lax.Precision.HIGHEST,
                           preferred_element_type=jnp.float32)


def _sigmoid(x):
    return 1.0 / (1.0 + jnp.exp(-x))


def _tc_dense(px0, py0, pw0, pm0, nc0, px1, py1, pw1, pm1, nc1,
              Wl1, bl1, Wr1, Wl2, bl2, Wr2, Wc1, bc1, Wc2, bc2):
    def body(px0r, py0r, pw0r, pm0r, nc0r, px1r, py1r, pw1r, pm1r, nc1r,
             Wl1r, bl1r, Wr1r, Wl2r, bl2r, Wr2r, Wc1r, bc1r, Wc2r, bc2r,
             outr):
        A1, B1 = Wl1r[...], bl1r[...]
        R1 = Wr1r[...]
        A2, B2, R2 = Wl2r[...], bl2r[...], Wr2r[...]

        def graph(pxr, pyr, pwr, pmr, ncr):
            nc = ncr[...][:G, 0:1]
            inv = 1.0 / jnp.maximum(nc, 1.0)
            u = jnp.where(nc > 0.0, 1.0, 0.0)
            px = pxr[...][:G, :] * inv
            py = pyr[...][:G, :] * inv
            pw = pwr[...][:G, :] * inv
            pm = pmr[...][:G, 0:1] * inv
            Pf = _mm(py, A1) + _mm(px, R1) + u * B1
            Pz = _mm(pw, A1) + _mm(py, R1) + pm * B1
            return _mm(Pz, A2) + _mm(Pf, R2) + u * B2

        e0 = graph(px0r, py0r, pw0r, pm0r, nc0r)
        e1 = graph(px1r, py1r, pw1r, pm1r, nc1r)
        comb = jnp.concatenate([e0, e1], axis=1)
        h = _sigmoid(_mm(comb, Wc1r[...]) + bc1r[...])
        o = jnp.sum(h * Wc2r[...], axis=1, keepdims=True) + bc2r[...][0, 0]
        outr[...] = _sigmoid(o)

    return pl.pallas_call(
        body,
        out_shape=jax.ShapeDtypeStruct((G, 1), jnp.float32),
    )(px0, py0, pw0, pm0, nc0, px1, py1, pw1, pm1, nc1,
      Wl1, bl1.reshape(1, -1), Wr1, Wl2, bl2.reshape(1, -1), Wr2,
      Wc1, bc1.reshape(1, -1), Wc2, bc2.reshape(1, 1))


def _prep_graph(x, edge_index, batch):
    src = edge_index[0].astype(jnp.int32)
    dst = edge_index[1].astype(jnp.int32)
    srcp = jnp.concatenate([src, jnp.zeros((EP - E,), jnp.int32)]).reshape(NSUB, ECH, CH)
    dstp = jnp.concatenate([dst, jnp.full((EP - E,), N, jnp.int32)]).reshape(NSUB, ECH, CH)
    xp = jnp.concatenate([x, jnp.zeros((NP - N, D), x.dtype)], axis=0)
    bflat = jnp.concatenate([batch.astype(jnp.int32),
                             jnp.full((NP - N,), G, jnp.int32)])
    bp = bflat.reshape(NSUB, NCH, CH)
    return xp, srcp, dstp, bp


def kernel(x0, edge_index0, batch0, x1, edge_index1, batch1,
           Wl1, bl1, Wr1, Wl2, bl2, Wr2, Wc1, bc1, Wc2, bc2):
    x0p, s0, d0, b0 = _prep_graph(x0, edge_index0, batch0)
    x1p, s1, d1, b1 = _prep_graph(x1, edge_index1, batch1)
    z128 = jnp.zeros((CH, D), jnp.float32)
    ones128 = jnp.ones((CH, D), jnp.float32)

    hbm = lambda a: pltpu.with_memory_space_constraint(a, pltpu.HBM)
    (_, _, px0, py0, pw0, pm0, nc0, px1, py1, pw1, pm1, nc1) = _sc_passes(
        hbm(x0p), hbm(x1p), hbm(s0), hbm(d0), hbm(s1), hbm(d1),
        hbm(b0), hbm(b1), hbm(z128), hbm(ones128))

    prob = _tc_dense(px0, py0, pw0, pm0, nc0, px1, py1, pw1, pm1, nc1,
                     Wl1, bl1, Wr1, Wl2, bl2, Wr2, Wc1, bc1, Wc2, bc2)
    return jnp.squeeze(prob, axis=-1)
